# Initial kernel scaffold; baseline (speedup 1.0000x reference)
import jax
import jax.numpy as jnp
from jax.experimental import pallas as pl


def kernel(inputs, params):
    raise NotImplementedError("write your pallas kernel here")


# speedup vs baseline: 5.4627x; 5.4627x over previous
import functools
import jax
import jax.numpy as jnp
import numpy as np
from jax.experimental import pallas as pl
from jax.experimental.pallas import tpu as pltpu
from jax.experimental.pallas import tpu_sc as plsc

B, C, N = 2, 6, 4096
K_CLS = 13
PLANES = [32, 64]
NSAMPLE = [8, 16]
SHARE = 8
N2 = N // 4
EPS = 1e-5
NEG_BIG = 3.0e38


def _bn_scale_shift(p, s1, s2, count):
    s1 = s1.reshape(-1)
    s2 = s2.reshape(-1)
    m = s1 / count
    v = s2 / count - m * m
    s = p["g"] * jax.lax.rsqrt(v + EPS)
    t = p["b"] - m * s
    return s, t


def _row2(a):
    return a.reshape(1, -1)


def _gather_rows(table, idx):
    return jnp.take(table, idx, axis=0)


def _dense(X, W, b, pre=None, want_stats=True, emit_a=False, bm=2048):
    M, din = X.shape
    dout = W.shape[1]
    bm = min(bm, M)
    grid = M // bm
    assert M % bm == 0

    def body(*refs):
        i = pl.program_id(0)
        ir = iter(refs)
        x_ref = next(ir)
        w_ref = next(ir)
        b_ref = next(ir)
        if pre is not None:
            s_ref = next(ir)
            t_ref = next(ir)
        y_ref = next(ir)
        a_ref = next(ir) if emit_a else None
        if want_stats:
            s1_ref = next(ir)
            s2_ref = next(ir)
        x = x_ref[...]
        if pre is not None:
            x = jnp.maximum(x * s_ref[...] + t_ref[...], 0.0)
        if emit_a:
            a_ref[...] = x
        y = jnp.dot(x, w_ref[...], preferred_element_type=jnp.float32) + b_ref[...]
        y_ref[...] = y
        if want_stats:
            @pl.when(i == 0)
            def _():
                s1_ref[...] = jnp.zeros_like(s1_ref)
                s2_ref[...] = jnp.zeros_like(s2_ref)
            s1_ref[...] += jnp.sum(y, 0, keepdims=True)
            s2_ref[...] += jnp.sum(y * y, 0, keepdims=True)

    in_specs = [
        pl.BlockSpec((bm, din), lambda i: (i, 0)),
        pl.BlockSpec((din, dout), lambda i: (0, 0)),
        pl.BlockSpec((1, dout), lambda i: (0, 0)),
    ]
    args = [X, W, _row2(b)]
    if pre is not None:
        in_specs += [pl.BlockSpec((1, din), lambda i: (0, 0))] * 2
        args += [_row2(pre[0]), _row2(pre[1])]
    out_specs = [pl.BlockSpec((bm, dout), lambda i: (i, 0))]
    out_shapes = [jax.ShapeDtypeStruct((M, dout), jnp.float32)]
    if emit_a:
        out_specs.append(pl.BlockSpec((bm, din), lambda i: (i, 0)))
        out_shapes.append(jax.ShapeDtypeStruct((M, din), jnp.float32))
    if want_stats:
        out_specs += [pl.BlockSpec((1, dout), lambda i: (0, 0))] * 2
        out_shapes += [jax.ShapeDtypeStruct((1, dout), jnp.float32)] * 2
    outs = pl.pallas_call(
        body, grid=(grid,), in_specs=in_specs, out_specs=out_specs,
        out_shape=out_shapes)(*args)
    outs = list(outs)
    y = outs.pop(0)
    a = outs.pop(0) if emit_a else None
    st = (outs[0], outs[1]) if want_stats else None
    res = [y]
    if emit_a:
        res.append(a)
    if want_stats:
        res.append(st)
    return res


def _ew_bnrelu(X, s, t, add=None, bm=2048):
    M, d = X.shape
    bm = min(bm, M)

    def body(*refs):
        if add is not None:
            x_ref, s_ref, t_ref, a_ref, o_ref = refs
        else:
            x_ref, s_ref, t_ref, o_ref = refs
        y = x_ref[...] * s_ref[...] + t_ref[...]
        if add is not None:
            y = y + a_ref[...]
        o_ref[...] = jnp.maximum(y, 0.0)

    in_specs = [pl.BlockSpec((bm, d), lambda i: (i, 0)),
                pl.BlockSpec((1, d), lambda i: (0, 0)),
                pl.BlockSpec((1, d), lambda i: (0, 0))]
    args = [X, _row2(s), _row2(t)]
    if add is not None:
        in_specs.append(pl.BlockSpec((bm, d), lambda i: (i, 0)))
        args.append(add)
    return pl.pallas_call(
        body, grid=(M // bm,), in_specs=in_specs,
        out_specs=pl.BlockSpec((bm, d), lambda i: (i, 0)),
        out_shape=jax.ShapeDtypeStruct((M, d), jnp.float32))(*args)


def _knn_pallas(qpos, rpos, k, bq=256):
    Bq, Mq, _ = qpos.shape
    Mr = rpos.shape[1]
    qp = jnp.concatenate([qpos, jnp.zeros((Bq, Mq, 13), jnp.float32)], -1)
    rt = jnp.transpose(rpos, (0, 2, 1))
    rt = jnp.concatenate([rt, jnp.zeros((Bq, 5, Mr), jnp.float32)], 1)

    def body(q_ref, r_ref, idx_ref):
        q = q_ref[0]
        r = r_ref[0]
        q3 = q[:, :3]
        r3 = r[:3, :]
        qq = jnp.sum(q3 * q3, 1, keepdims=True)
        rr = jnp.sum(r3 * r3, 0, keepdims=True)
        cross = jnp.dot(q3, r3, preferred_element_type=jnp.float32)
        d = qq - 2.0 * cross + rr
        iota_r = jax.lax.broadcasted_iota(jnp.int32, (bq, Mr), 1)
        for j in range(k):
            m = jnp.min(d, 1, keepdims=True)
            am = jnp.min(jnp.where(d == m, iota_r, Mr), 1, keepdims=True)
            idx_ref[0, :, pl.ds(j, 1)] = am
            d = jnp.where(iota_r == am, NEG_BIG, d)

    return pl.pallas_call(
        body, grid=(Bq, Mq // bq),
        in_specs=[pl.BlockSpec((1, bq, 16), lambda b, i: (b, i, 0)),
                  pl.BlockSpec((1, 8, Mr), lambda b, i: (b, 0, 0))],
        out_specs=pl.BlockSpec((1, bq, k), lambda b, i: (b, i, 0)),
        out_shape=jax.ShapeDtypeStruct((Bq, Mq, k), jnp.int32))(qp, rt)


def _fps_pallas(p1):
    SUB, LN = 8, N // 8
    pos = jnp.transpose(p1, (0, 2, 1)).reshape(B, 3, SUB, LN)

    def body(p_ref, o_ref):
        xr = p_ref[0, 0]
        yr = p_ref[0, 1]
        zr = p_ref[0, 2]
        ii = (jax.lax.broadcasted_iota(jnp.int32, (SUB, LN), 0) * LN
              + jax.lax.broadcasted_iota(jnp.int32, (SUB, LN), 1))
        ii_out = (jax.lax.broadcasted_iota(jnp.int32, (8, N2 // 8), 0) * (N2 // 8)
                  + jax.lax.broadcasted_iota(jnp.int32, (8, N2 // 8), 1))

        def step(i, st):
            dist, idx_arr, last = st
            sel = ii == last
            lx = jnp.sum(jnp.where(sel, xr, 0.0))
            ly = jnp.sum(jnp.where(sel, yr, 0.0))
            lz = jnp.sum(jnp.where(sel, zr, 0.0))
            d = (xr - lx) ** 2 + (yr - ly) ** 2 + (zr - lz) ** 2
            dist = jnp.minimum(dist, d)
            m = jnp.max(dist)
            g = jnp.min(jnp.where(dist == m, ii, N))
            idx_arr = jnp.where(ii_out == i, g, idx_arr)
            return dist, idx_arr, g

        dist0 = jnp.full((SUB, LN), 1e10, jnp.float32)
        idx0 = jnp.zeros((8, N2 // 8), jnp.int32)
        _, idx_arr, _ = jax.lax.fori_loop(1, N2, step, (dist0, idx0, 0))
        o_ref[0] = idx_arr

    out = pl.pallas_call(
        body, grid=(B,),
        in_specs=[pl.BlockSpec((1, 3, SUB, LN), lambda b: (b, 0, 0, 0))],
        out_specs=pl.BlockSpec((1, 8, N2 // 8), lambda b: (b, 0, 0)),
        out_shape=jax.ShapeDtypeStruct((B, 8, N2 // 8), jnp.int32))(pos)
    return out.reshape(B, N2)


def _pe_pre_kernel(gpos, posf, W, b, ns, bm=512):
    M = posf.shape[0]
    bm = min(bm, M)
    Wp = jnp.zeros((16, 16), jnp.float32).at[:3, :3].set(W)
    bp = jnp.zeros((16,), jnp.float32).at[:3].set(b)

    def body(g_ref, p_ref, w_ref, b_ref, o_ref, s1_ref, s2_ref):
        i = pl.program_id(0)
        g = g_ref[...].reshape(bm, ns, 16)
        p = p_ref[...]
        pr = g - p[:, None, :]
        pr = pr.reshape(bm * ns, 16)
        y = jnp.dot(pr, w_ref[...], preferred_element_type=jnp.float32) + b_ref[...]
        o_ref[...] = y
        @pl.when(i == 0)
        def _():
            s1_ref[...] = jnp.zeros_like(s1_ref)
            s2_ref[...] = jnp.zeros_like(s2_ref)
        s1_ref[...] += jnp.sum(y, 0, keepdims=True)
        s2_ref[...] += jnp.sum(y * y, 0, keepdims=True)

    outs = pl.pallas_call(
        body, grid=(M // bm,),
        in_specs=[pl.BlockSpec((bm * ns, 16), lambda i: (i, 0)),
                  pl.BlockSpec((bm, 16), lambda i: (i, 0)),
                  pl.BlockSpec((16, 16), lambda i: (0, 0)),
                  pl.BlockSpec((1, 16), lambda i: (0, 0))],
        out_specs=[pl.BlockSpec((bm * ns, 16), lambda i: (i, 0)),
                   pl.BlockSpec((1, 16), lambda i: (0, 0)),
                   pl.BlockSpec((1, 16), lambda i: (0, 0))],
        out_shape=[jax.ShapeDtypeStruct((M * ns, 16), jnp.float32),
                   jax.ShapeDtypeStruct((1, 16), jnp.float32),
                   jax.ShapeDtypeStruct((1, 16), jnp.float32)])(
        gpos, posf, Wp, _row2(bp))
    return outs[0], (outs[1][:, :3], outs[2][:, :3])


def _pe_w_kernel(pe_pre, x_k, q, s, t, W2, b2, ns, c, bm=512):
    M = q.shape[0]
    bm = min(bm, M)
    sp = jnp.zeros((16,), jnp.float32).at[:3].set(s)
    tp = jnp.zeros((16,), jnp.float32).at[:3].set(t)
    W2p = jnp.zeros((16, c), jnp.float32).at[:3, :].set(W2)

    def body(pp_ref, xk_ref, q_ref, s_ref, t_ref, w_ref, b_ref,
             pe_ref, wr_ref, s1_ref, s2_ref):
        i = pl.program_id(0)
        a = jnp.maximum(pp_ref[...] * s_ref[...] + t_ref[...], 0.0)
        pe = jnp.dot(a, w_ref[...], preferred_element_type=jnp.float32) + b_ref[...]
        pe_ref[...] = pe
        w = (xk_ref[...].reshape(bm, ns, c) - q_ref[...][:, None, :]).reshape(
            bm * ns, c) + pe
        wr_ref[...] = w
        @pl.when(i == 0)
        def _():
            s1_ref[...] = jnp.zeros_like(s1_ref)
            s2_ref[...] = jnp.zeros_like(s2_ref)
        s1_ref[...] += jnp.sum(w, 0, keepdims=True)
        s2_ref[...] += jnp.sum(w * w, 0, keepdims=True)

    outs = pl.pallas_call(
        body, grid=(M // bm,),
        in_specs=[pl.BlockSpec((bm * ns, 16), lambda i: (i, 0)),
                  pl.BlockSpec((bm * ns, c), lambda i: (i, 0)),
                  pl.BlockSpec((bm, c), lambda i: (i, 0)),
                  pl.BlockSpec((1, 16), lambda i: (0, 0)),
                  pl.BlockSpec((1, 16), lambda i: (0, 0)),
                  pl.BlockSpec((16, c), lambda i: (0, 0)),
                  pl.BlockSpec((1, c), lambda i: (0, 0))],
        out_specs=[pl.BlockSpec((bm * ns, c), lambda i: (i, 0)),
                   pl.BlockSpec((bm * ns, c), lambda i: (i, 0)),
                   pl.BlockSpec((1, c), lambda i: (0, 0)),
                   pl.BlockSpec((1, c), lambda i: (0, 0))],
        out_shape=[jax.ShapeDtypeStruct((M * ns, c), jnp.float32),
                   jax.ShapeDtypeStruct((M * ns, c), jnp.float32),
                   jax.ShapeDtypeStruct((1, c), jnp.float32),
                   jax.ShapeDtypeStruct((1, c), jnp.float32)])(
        pe_pre, x_k, q, _row2(sp), _row2(tp), W2p, _row2(b2))
    return outs[0], outs[1], (outs[2], outs[3])


def _attn_out_kernel(w1, x_v, pe, s, t, W2, b2, ns, c, bm=512):
    M = x_v.shape[0] // ns
    bm = min(bm, M)
    cs = c // SHARE

    def body(w1_ref, xv_ref, pe_ref, s_ref, t_ref, w_ref, b_ref,
             o_ref, s1_ref, s2_ref):
        i = pl.program_id(0)
        a = jnp.maximum(w1_ref[...] * s_ref[...] + t_ref[...], 0.0)
        w2 = jnp.dot(a, w_ref[...], preferred_element_type=jnp.float32) + b_ref[...]
        w3 = w2.reshape(bm, ns, cs)
        m = jnp.max(w3, axis=1, keepdims=True)
        e = jnp.exp(w3 - m)
        sm = e / jnp.sum(e, axis=1, keepdims=True)
        smf = jnp.concatenate([sm] * SHARE, axis=-1)
        xvpe = (xv_ref[...] + pe_ref[...]).reshape(bm, ns, c)
        out = jnp.sum(xvpe * smf, axis=1)
        o_ref[...] = out
        @pl.when(i == 0)
        def _():
            s1_ref[...] = jnp.zeros_like(s1_ref)
            s2_ref[...] = jnp.zeros_like(s2_ref)
        s1_ref[...] += jnp.sum(out, 0, keepdims=True)
        s2_ref[...] += jnp.sum(out * out, 0, keepdims=True)

    outs = pl.pallas_call(
        body, grid=(M // bm,),
        in_specs=[pl.BlockSpec((bm * ns, cs), lambda i: (i, 0)),
                  pl.BlockSpec((bm * ns, c), lambda i: (i, 0)),
                  pl.BlockSpec((bm * ns, c), lambda i: (i, 0)),
                  pl.BlockSpec((1, cs), lambda i: (0, 0)),
                  pl.BlockSpec((1, cs), lambda i: (0, 0)),
                  pl.BlockSpec((cs, cs), lambda i: (0, 0)),
                  pl.BlockSpec((1, cs), lambda i: (0, 0))],
        out_specs=[pl.BlockSpec((bm, c), lambda i: (i, 0)),
                   pl.BlockSpec((1, c), lambda i: (0, 0)),
                   pl.BlockSpec((1, c), lambda i: (0, 0))],
        out_shape=[jax.ShapeDtypeStruct((M, c), jnp.float32),
                   jax.ShapeDtypeStruct((1, c), jnp.float32),
                   jax.ShapeDtypeStruct((1, c), jnp.float32)])(
        w1, x_v, pe, _row2(s), _row2(t), W2, _row2(b2))
    return outs[0], (outs[1], outs[2])


def _td2_feat_kernel(gp, gx, p2f, Wrel, Wx, b, ns, bm=256):
    M = p2f.shape[0]
    bm = min(bm, M)
    dout = Wx.shape[1]
    Wr = jnp.zeros((16, dout), jnp.float32).at[:3, :].set(Wrel)

    def body(gp_ref, gx_ref, p_ref, wr_ref, wx_ref, b_ref, o_ref, s1_ref, s2_ref):
        i = pl.program_id(0)
        rel = (gp_ref[...].reshape(bm, ns, 16) - p_ref[...][:, None, :]).reshape(
            bm * ns, 16)
        y = (jnp.dot(rel, wr_ref[...], preferred_element_type=jnp.float32)
             + jnp.dot(gx_ref[...], wx_ref[...], preferred_element_type=jnp.float32)
             + b_ref[...])
        o_ref[...] = y
        @pl.when(i == 0)
        def _():
            s1_ref[...] = jnp.zeros_like(s1_ref)
            s2_ref[...] = jnp.zeros_like(s2_ref)
        s1_ref[...] += jnp.sum(y, 0, keepdims=True)
        s2_ref[...] += jnp.sum(y * y, 0, keepdims=True)

    din = gx.shape[1]
    outs = pl.pallas_call(
        body, grid=(M // bm,),
        in_specs=[pl.BlockSpec((bm * ns, 16), lambda i: (i, 0)),
                  pl.BlockSpec((bm * ns, din), lambda i: (i, 0)),
                  pl.BlockSpec((bm, 16), lambda i: (i, 0)),
                  pl.BlockSpec((16, dout), lambda i: (0, 0)),
                  pl.BlockSpec((din, dout), lambda i: (0, 0)),
                  pl.BlockSpec((1, dout), lambda i: (0, 0))],
        out_specs=[pl.BlockSpec((bm * ns, dout), lambda i: (i, 0)),
                   pl.BlockSpec((1, dout), lambda i: (0, 0)),
                   pl.BlockSpec((1, dout), lambda i: (0, 0))],
        out_shape=[jax.ShapeDtypeStruct((M * ns, dout), jnp.float32),
                   jax.ShapeDtypeStruct((1, dout), jnp.float32),
                   jax.ShapeDtypeStruct((1, dout), jnp.float32)])(
        gp, gx, p2f, Wr, Wx, _row2(b))
    return outs[0], (outs[1], outs[2])


def _td2_max_kernel(f_pre, s, t, ns, bm=256):
    Mns, d = f_pre.shape
    M = Mns // ns
    bm = min(bm, M)

    def body(f_ref, s_ref, t_ref, o_ref):
        a = jnp.maximum(f_ref[...] * s_ref[...] + t_ref[...], 0.0)
        o_ref[...] = jnp.max(a.reshape(bm, ns, d), axis=1)

    return pl.pallas_call(
        body, grid=(M // bm,),
        in_specs=[pl.BlockSpec((bm * ns, d), lambda i: (i, 0)),
                  pl.BlockSpec((1, d), lambda i: (0, 0)),
                  pl.BlockSpec((1, d), lambda i: (0, 0))],
        out_specs=pl.BlockSpec((bm, d), lambda i: (i, 0)),
        out_shape=jax.ShapeDtypeStruct((M, d), jnp.float32))(
        f_pre, _row2(s), _row2(t))


def _dec2_pre_kernel(x2b, W2, b2, W1a, W1b, b1):
    c = x2b.shape[1]

    def body(x_ref, w2_ref, b2_ref, wa_ref, wb_ref, b1_ref, o_ref, s1_ref, s2_ref):
        s1_ref[...] = jnp.zeros_like(s1_ref)
        s2_ref[...] = jnp.zeros_like(s2_ref)
        for bb in range(B):
            xb = x_ref[pl.ds(bb * N2, N2), :]
            gm = jnp.sum(xb, 0, keepdims=True) / N2
            g2 = jnp.maximum(jnp.dot(gm, w2_ref[...],
                                     preferred_element_type=jnp.float32)
                             + b2_ref[...], 0.0)
            y = (jnp.dot(xb, wa_ref[...], preferred_element_type=jnp.float32)
                 + jnp.dot(g2, wb_ref[...], preferred_element_type=jnp.float32)
                 + b1_ref[...])
            o_ref[pl.ds(bb * N2, N2), :] = y
            s1_ref[...] += jnp.sum(y, 0, keepdims=True)
            s2_ref[...] += jnp.sum(y * y, 0, keepdims=True)

    outs = pl.pallas_call(
        body,
        in_specs=[pl.BlockSpec(x2b.shape, lambda: (0, 0))] +
                 [pl.BlockSpec(a.shape, lambda: (0, 0)) for a in
                  (W2, _row2(b2), W1a, W1b, _row2(b1))],
        out_specs=[pl.BlockSpec((B * N2, c), lambda: (0, 0)),
                   pl.BlockSpec((1, c), lambda: (0, 0)),
                   pl.BlockSpec((1, c), lambda: (0, 0))],
        out_shape=[jax.ShapeDtypeStruct((B * N2, c), jnp.float32),
                   jax.ShapeDtypeStruct((1, c), jnp.float32),
                   jax.ShapeDtypeStruct((1, c), jnp.float32)])(
        x2b, W2, _row2(b2), W1a, W1b, _row2(b1))
    return outs[0], (outs[1], outs[2])


def _interp_kernel(a_pre, sa, ta, gpi, p1f, gxi, bm=1024):
    M, c = a_pre.shape
    bm = min(bm, M)

    def body(a_ref, s_ref, t_ref, gp_ref, p_ref, gx_ref, o_ref):
        a = jnp.maximum(a_ref[...] * s_ref[...] + t_ref[...], 0.0)
        gp = gp_ref[...].reshape(bm, 4, 16)[:, :3, :3]
        diff = gp - p_ref[...][:, None, :3]
        d = jnp.sqrt(jnp.sum(diff * diff, axis=-1)) + 1e-8
        w = 1.0 / d
        w = w / jnp.sum(w, -1, keepdims=True)
        gx = gx_ref[...].reshape(bm, 4, c)[:, :3, :]
        o_ref[...] = a + jnp.sum(gx * w[:, :, None], axis=1)

    return pl.pallas_call(
        body, grid=(M // bm,),
        in_specs=[pl.BlockSpec((bm, c), lambda i: (i, 0)),
                  pl.BlockSpec((1, c), lambda i: (0, 0)),
                  pl.BlockSpec((1, c), lambda i: (0, 0)),
                  pl.BlockSpec((bm * 4, 16), lambda i: (i, 0)),
                  pl.BlockSpec((bm, 16), lambda i: (i, 0)),
                  pl.BlockSpec((bm * 4, c), lambda i: (i, 0))],
        out_specs=pl.BlockSpec((bm, c), lambda i: (i, 0)),
        out_shape=jax.ShapeDtypeStruct((M, c), jnp.float32))(
        a_pre, _row2(sa), _row2(ta), gpi, p1f, gxi)


def _heads_kernel(hh_pre, s, t, Wc, bc, We, be, bm=2048):
    M = hh_pre.shape[0]
    bm = min(bm, M)

    def body(h_ref, s_ref, t_ref, wc_ref, bc_ref, we_ref, be_ref, c_ref, e_ref):
        h = jnp.maximum(h_ref[...] * s_ref[...] + t_ref[...], 0.0)
        c_ref[...] = jnp.dot(h[:, :32], wc_ref[...],
                             preferred_element_type=jnp.float32) + bc_ref[...]
        e_ref[...] = jnp.dot(h[:, 32:], we_ref[...],
                             preferred_element_type=jnp.float32) + be_ref[...]

    return pl.pallas_call(
        body, grid=(M // bm,),
        in_specs=[pl.BlockSpec((bm, 64), lambda i: (i, 0)),
                  pl.BlockSpec((1, 64), lambda i: (0, 0)),
                  pl.BlockSpec((1, 64), lambda i: (0, 0)),
                  pl.BlockSpec((32, K_CLS), lambda i: (0, 0)),
                  pl.BlockSpec((1, K_CLS), lambda i: (0, 0)),
                  pl.BlockSpec((32, 2), lambda i: (0, 0)),
                  pl.BlockSpec((1, 2), lambda i: (0, 0))],
        out_specs=[pl.BlockSpec((bm, K_CLS), lambda i: (i, 0)),
                   pl.BlockSpec((bm, 2), lambda i: (i, 0))],
        out_shape=[jax.ShapeDtypeStruct((M, K_CLS), jnp.float32),
                   jax.ShapeDtypeStruct((M, 2), jnp.float32)])(
        hh_pre, _row2(s), _row2(t), Wc, _row2(bc), We, _row2(be))


def _pt_layer(p, posf16, gpos16, q, kf, v, idx, ns, c):
    M = q.shape[0]
    pe_pre, st = _pe_pre_kernel(gpos16, posf16, p["p1"]["W"], p["p1"]["b"], ns)
    s, t = _bn_scale_shift(p["pbn"], st[0], st[1], M * ns)

    kv = jnp.concatenate([kf, v], axis=1)
    gkv = _gather_rows(kv, idx)
    x_k = gkv[:, :c]
    x_v = gkv[:, c:]

    pe, w_raw, st = _pe_w_kernel(pe_pre, x_k, q, s, t,
                                 p["p2"]["W"], p["p2"]["b"], ns, c)
    s, t = _bn_scale_shift(p["wbn1"], st[0], st[1], M * ns)
    w1, st = _dense(w_raw, p["w1"]["W"], p["w1"]["b"], pre=(s, t))
    s, t = _bn_scale_shift(p["wbn2"], st[0], st[1], M * ns)
    out, st = _attn_out_kernel(w1, x_v, pe, s, t,
                               p["w2"]["W"], p["w2"]["b"], ns, c)
    return out, st


def _pt_block(p, posf16, gpos16, x, idx, ns, c):
    h_pre, st = _dense(x, p["l1"]["W"], p["l1"]["b"])
    return _pt_block_from(p, posf16, gpos16, x, h_pre, st, idx, ns, c)


def _pad16(a):
    return jnp.concatenate([a, jnp.zeros(a.shape[:-1] + (16 - a.shape[-1],),
                                         jnp.float32)], -1)


def _forward(inputs, params):
    pxo = jnp.transpose(inputs, (0, 2, 1))
    x0 = pxo.reshape(B * N, C)
    p1 = pxo[:, :, :3]
    p1f = p1.reshape(B * N, 3)
    p1f16 = _pad16(p1f)

    td = params["enc1_td"]
    y_pre, st = _dense(x0, td["lin"]["W"], td["lin"]["b"])
    s, t = _bn_scale_shift(td["bn"], st[0], st[1], B * N)

    idx1 = _knn_pallas(p1, p1, NSAMPLE[0])
    gidx1 = (idx1 + (jnp.arange(B, dtype=jnp.int32) * N)[:, None, None]).reshape(-1)
    gpos1 = _gather_rows(p1f16, gidx1)

    blk = params["enc1_blk"]
    h_pre, x1, st1 = _dense(y_pre, blk["l1"]["W"], blk["l1"]["b"],
                            pre=(s, t), emit_a=True)
    x1b = _pt_block_from(blk, p1f16, gpos1, x1, h_pre, st1, gidx1,
                         NSAMPLE[0], PLANES[0])

    sidx = _fps_pallas(p1)
    gsidx = (sidx + (jnp.arange(B, dtype=jnp.int32) * N)[:, None]).reshape(-1)
    p2f = _gather_rows(p1f, gsidx)
    p2 = p2f.reshape(B, N2, 3)
    p2f16 = _pad16(p2f)
    nidx = _knn_pallas(p2, p1, NSAMPLE[1])
    gnidx = (nidx + (jnp.arange(B, dtype=jnp.int32) * N)[:, None, None]).reshape(-1)
    gp = _gather_rows(p1f16, gnidx)
    gx = _gather_rows(x1b, gnidx)
    td = params["enc2_td"]
    f_pre, st = _td2_feat_kernel(gp, gx, p2f16, td["lin"]["W"][:3],
                                 td["lin"]["W"][3:], td["lin"]["b"], NSAMPLE[1])
    s, t = _bn_scale_shift(td["bn"], st[0], st[1], B * N2 * NSAMPLE[1])
    x2 = _td2_max_kernel(f_pre, s, t, NSAMPLE[1])

    idx2 = _knn_pallas(p2, p2, NSAMPLE[1])
    gidx2 = (idx2 + (jnp.arange(B, dtype=jnp.int32) * N2)[:, None, None]).reshape(-1)
    gpos2 = _gather_rows(p2f16, gidx2)

    x2b = _pt_block(params["enc2_blk"], p2f16, gpos2, x2, gidx2,
                    NSAMPLE[1], PLANES[1])

    up = params["dec2_up"]
    h2_pre, st = _dec2_pre_kernel(x2b, up["l2"]["W"], up["l2"]["b"],
                                  up["l1"]["W"][:PLANES[1]],
                                  up["l1"]["W"][PLANES[1]:], up["l1"]["b"])
    s, t = _bn_scale_shift(up["bn1"], st[0], st[1], B * N2)
    h2 = _ew_bnrelu(h2_pre, s, t)
    x2d = _pt_block(params["dec2_blk"], p2f16, gpos2, h2, gidx2,
                    NSAMPLE[1], PLANES[1])

    up = params["dec1_up"]
    a_pre, sta = _dense(x1b, up["l1"]["W"], up["l1"]["b"])
    sa, ta = _bn_scale_shift(up["bn1"], sta[0], sta[1], B * N)
    b_pre, stb = _dense(x2d, up["l2"]["W"], up["l2"]["b"])
    sb, tb = _bn_scale_shift(up["bn2"], stb[0], stb[1], B * N2)
    bfeat = _ew_bnrelu(b_pre, sb, tb)

    iidx = _knn_pallas(p1, p2, 3)
    iidx4 = jnp.concatenate([iidx, iidx[:, :, :1]], axis=-1)
    giidx = (iidx4 + (jnp.arange(B, dtype=jnp.int32) * N2)[:, None, None]
             ).reshape(-1)
    gpi = _gather_rows(p2f16, giidx)
    gxi = _gather_rows(bfeat, giidx)
    h1 = _interp_kernel(a_pre, sa, ta, gpi, p1f16, gxi)

    x1d = _pt_block(params["dec1_blk"], p1f16, gpos1, h1, gidx1,
                    NSAMPLE[0], PLANES[0])

    ch, eh = params["cls_head"], params["edge_head"]
    Wcat = jnp.concatenate([ch["l1"]["W"], eh["l1"]["W"]], axis=1)
    bcat = jnp.concatenate([ch["l1"]["b"], eh["l1"]["b"]])
    hh_pre, st = _dense(x1d, Wcat, bcat)
    sA, tA = _bn_scale_shift(ch["bn"], st[0][:, :32], st[1][:, :32], B * N)
    sB, tB = _bn_scale_shift(eh["bn"], st[0][:, 32:], st[1][:, 32:], B * N)
    s = jnp.concatenate([sA, sB])
    t = jnp.concatenate([tA, tB])
    cls, edge = _heads_kernel(hh_pre, s, t, ch["l2"]["W"], ch["l2"]["b"],
                              eh["l2"]["W"], eh["l2"]["b"])
    cls = jnp.transpose(cls.reshape(B, N, K_CLS), (0, 2, 1))
    edge = jnp.transpose(edge.reshape(B, N, 2), (0, 2, 1))
    return (cls, edge)


def _pt_block_from(p, posf16, gpos16, x, h_pre, st, idx, ns, c):
    M = x.shape[0]
    s, t = _bn_scale_shift(p["bn1"], st[0], st[1], M)
    tr = p["tr"]
    Wqkv = jnp.concatenate([tr["q"]["W"], tr["k"]["W"], tr["v"]["W"]], axis=1)
    bqkv = jnp.concatenate([tr["q"]["b"], tr["k"]["b"], tr["v"]["b"]])
    (qkv,) = _dense(h_pre, Wqkv, bqkv, pre=(s, t), want_stats=False)
    q, kf, v = qkv[:, :c], qkv[:, c:2 * c], qkv[:, 2 * c:]
    tt, st = _pt_layer(tr, posf16, gpos16, q, kf, v, idx, ns, c)
    s, t = _bn_scale_shift(p["bn2"], st[0], st[1], M)
    h3_pre, st = _dense(tt, p["l3"]["W"], p["l3"]["b"], pre=(s, t))
    s, t = _bn_scale_shift(p["bn3"], st[0], st[1], M)
    return _ew_bnrelu(h3_pre, s, t, add=x)


def kernel(inputs, params):
    return _forward(inputs, params)


# speedup vs baseline: 8.4104x; 1.5396x over previous
import functools
import jax
import jax.numpy as jnp
import numpy as np
from jax.experimental import pallas as pl
from jax.experimental.pallas import tpu as pltpu
from jax.experimental.pallas import tpu_sc as plsc

B, C, N = 2, 6, 4096
K_CLS = 13
PLANES = [32, 64]
NSAMPLE = [8, 16]
SHARE = 8
N2 = N // 4
EPS = 1e-5
NEG_BIG = 3.0e38


def _bn_scale_shift(p, s1, s2, count):
    s1 = s1.reshape(-1)
    s2 = s2.reshape(-1)
    m = s1 / count
    v = s2 / count - m * m
    s = p["g"] * jax.lax.rsqrt(v + EPS)
    t = p["b"] - m * s
    return s, t


def _row2(a):
    return a.reshape(1, -1)


def _gather_rows(table, idx):
    R, D = table.shape
    Rout = idx.shape[0]
    NW = 32
    assert Rout % NW == 0 and D % 16 == 0
    per_w = Rout // NW
    ch = min(128, per_w)
    assert per_w % ch == 0
    n_chunks = per_w // ch

    mesh = plsc.VectorSubcoreMesh(core_axis_name="c", subcore_axis_name="s")

    @functools.partial(
        pl.kernel, mesh=mesh,
        out_type=jax.ShapeDtypeStruct((Rout, D), jnp.float32),
        compiler_params=pltpu.CompilerParams(use_tc_tiling_on_sc=False),
        scratch_types=[
            pltpu.VMEM((per_w,), jnp.int32),
            pltpu.VMEM((ch, D), jnp.float32),
            pltpu.VMEM((ch, D), jnp.float32),
            pltpu.SemaphoreType.DMA,
            pltpu.SemaphoreType.DMA,
        ],
    )
    def k(table_hbm, idx_hbm, out_hbm, idx_v, rows0, rows1, sem0, sem1):
        wid = jax.lax.axis_index("s") * 2 + jax.lax.axis_index("c")
        base = wid * per_w
        pltpu.sync_copy(idx_hbm.at[pl.ds(base, per_w)], idx_v)
        bufs = [(rows0, sem0), (rows1, sem1)]
        cps = [None, None]
        for ci in range(n_chunks):
            rv, sem = bufs[ci % 2]
            cps[ci % 2] = pltpu.async_copy(
                table_hbm.at[idx_v.at[pl.ds(ci * ch, ch)]], rv, sem)
            if ci > 0:
                pv, psem = bufs[(ci - 1) % 2]
                cps[(ci - 1) % 2].wait()
                pltpu.sync_copy(pv, out_hbm.at[pl.ds(base + (ci - 1) * ch, ch)])
        lv, lsem = bufs[(n_chunks - 1) % 2]
        cps[(n_chunks - 1) % 2].wait()
        pltpu.sync_copy(lv, out_hbm.at[pl.ds(base + (n_chunks - 1) * ch, ch)])

    return k(table, idx)


def _dense(X, W, b, pre=None, want_stats=True, emit_a=False, bm=2048):
    M, din = X.shape
    dout = W.shape[1]
    bm = min(bm, M)
    grid = M // bm
    assert M % bm == 0

    def body(*refs):
        i = pl.program_id(0)
        ir = iter(refs)
        x_ref = next(ir)
        w_ref = next(ir)
        b_ref = next(ir)
        if pre is not None:
            s_ref = next(ir)
            t_ref = next(ir)
        y_ref = next(ir)
        a_ref = next(ir) if emit_a else None
        if want_stats:
            s1_ref = next(ir)
            s2_ref = next(ir)
        x = x_ref[...]
        if pre is not None:
            x = jnp.maximum(x * s_ref[...] + t_ref[...], 0.0)
        if emit_a:
            a_ref[...] = x
        y = jnp.dot(x, w_ref[...], preferred_element_type=jnp.float32) + b_ref[...]
        y_ref[...] = y
        if want_stats:
            @pl.when(i == 0)
            def _():
                s1_ref[...] = jnp.zeros_like(s1_ref)
                s2_ref[...] = jnp.zeros_like(s2_ref)
            s1_ref[...] += jnp.sum(y, 0, keepdims=True)
            s2_ref[...] += jnp.sum(y * y, 0, keepdims=True)

    in_specs = [
        pl.BlockSpec((bm, din), lambda i: (i, 0)),
        pl.BlockSpec((din, dout), lambda i: (0, 0)),
        pl.BlockSpec((1, dout), lambda i: (0, 0)),
    ]
    args = [X, W, _row2(b)]
    if pre is not None:
        in_specs += [pl.BlockSpec((1, din), lambda i: (0, 0))] * 2
        args += [_row2(pre[0]), _row2(pre[1])]
    out_specs = [pl.BlockSpec((bm, dout), lambda i: (i, 0))]
    out_shapes = [jax.ShapeDtypeStruct((M, dout), jnp.float32)]
    if emit_a:
        out_specs.append(pl.BlockSpec((bm, din), lambda i: (i, 0)))
        out_shapes.append(jax.ShapeDtypeStruct((M, din), jnp.float32))
    if want_stats:
        out_specs += [pl.BlockSpec((1, dout), lambda i: (0, 0))] * 2
        out_shapes += [jax.ShapeDtypeStruct((1, dout), jnp.float32)] * 2
    outs = pl.pallas_call(
        body, grid=(grid,), in_specs=in_specs, out_specs=out_specs,
        out_shape=out_shapes)(*args)
    outs = list(outs)
    y = outs.pop(0)
    a = outs.pop(0) if emit_a else None
    st = (outs[0], outs[1]) if want_stats else None
    res = [y]
    if emit_a:
        res.append(a)
    if want_stats:
        res.append(st)
    return res


def _ew_bnrelu(X, s, t, add=None, bm=2048):
    M, d = X.shape
    bm = min(bm, M)

    def body(*refs):
        if add is not None:
            x_ref, s_ref, t_ref, a_ref, o_ref = refs
        else:
            x_ref, s_ref, t_ref, o_ref = refs
        y = x_ref[...] * s_ref[...] + t_ref[...]
        if add is not None:
            y = y + a_ref[...]
        o_ref[...] = jnp.maximum(y, 0.0)

    in_specs = [pl.BlockSpec((bm, d), lambda i: (i, 0)),
                pl.BlockSpec((1, d), lambda i: (0, 0)),
                pl.BlockSpec((1, d), lambda i: (0, 0))]
    args = [X, _row2(s), _row2(t)]
    if add is not None:
        in_specs.append(pl.BlockSpec((bm, d), lambda i: (i, 0)))
        args.append(add)
    return pl.pallas_call(
        body, grid=(M // bm,), in_specs=in_specs,
        out_specs=pl.BlockSpec((bm, d), lambda i: (i, 0)),
        out_shape=jax.ShapeDtypeStruct((M, d), jnp.float32))(*args)


def _knn_pallas(qpos, rpos, k, bq=256):
    Bq, Mq, _ = qpos.shape
    Mr = rpos.shape[1]
    qp = jnp.concatenate([qpos, jnp.zeros((Bq, Mq, 13), jnp.float32)], -1)
    rt = jnp.transpose(rpos, (0, 2, 1))
    rt = jnp.concatenate([rt, jnp.zeros((Bq, 5, Mr), jnp.float32)], 1)

    def body(q_ref, r_ref, idx_ref):
        q = q_ref[0]
        r = r_ref[0]
        q3 = q[:, :3]
        r3 = r[:3, :]
        qq = jnp.sum(q3 * q3, 1, keepdims=True)
        rr = jnp.sum(r3 * r3, 0, keepdims=True)
        cross = jnp.dot(q3, r3, preferred_element_type=jnp.float32)
        d = qq - 2.0 * cross + rr
        iota_r = jax.lax.broadcasted_iota(jnp.int32, (bq, Mr), 1)
        for j in range(k):
            m = jnp.min(d, 1, keepdims=True)
            am = jnp.min(jnp.where(d == m, iota_r, Mr), 1, keepdims=True)
            idx_ref[0, :, pl.ds(j, 1)] = am
            d = jnp.where(iota_r == am, NEG_BIG, d)

    return pl.pallas_call(
        body, grid=(Bq, Mq // bq),
        in_specs=[pl.BlockSpec((1, bq, 16), lambda b, i: (b, i, 0)),
                  pl.BlockSpec((1, 8, Mr), lambda b, i: (b, 0, 0))],
        out_specs=pl.BlockSpec((1, bq, k), lambda b, i: (b, i, 0)),
        out_shape=jax.ShapeDtypeStruct((Bq, Mq, k), jnp.int32))(qp, rt)


def _fps_pallas(p1):
    SUB, LN = 8, N // 8
    pos = jnp.transpose(p1, (0, 2, 1)).reshape(B, 3, SUB, LN)

    def body(p_ref, o_ref):
        xr = p_ref[0, 0]
        yr = p_ref[0, 1]
        zr = p_ref[0, 2]
        ii = (jax.lax.broadcasted_iota(jnp.int32, (SUB, LN), 0) * LN
              + jax.lax.broadcasted_iota(jnp.int32, (SUB, LN), 1))
        ii_out = (jax.lax.broadcasted_iota(jnp.int32, (8, N2 // 8), 0) * (N2 // 8)
                  + jax.lax.broadcasted_iota(jnp.int32, (8, N2 // 8), 1))

        def step(i, st):
            dist, idx_arr, last = st
            sel = ii == last
            lx = jnp.sum(jnp.where(sel, xr, 0.0))
            ly = jnp.sum(jnp.where(sel, yr, 0.0))
            lz = jnp.sum(jnp.where(sel, zr, 0.0))
            d = (xr - lx) ** 2 + (yr - ly) ** 2 + (zr - lz) ** 2
            dist = jnp.minimum(dist, d)
            m = jnp.max(dist)
            g = jnp.min(jnp.where(dist == m, ii, N))
            idx_arr = jnp.where(ii_out == i, g, idx_arr)
            return dist, idx_arr, g

        dist0 = jnp.full((SUB, LN), 1e10, jnp.float32)
        idx0 = jnp.zeros((8, N2 // 8), jnp.int32)
        _, idx_arr, _ = jax.lax.fori_loop(1, N2, step, (dist0, idx0, 0))
        o_ref[0] = idx_arr

    out = pl.pallas_call(
        body, grid=(B,),
        in_specs=[pl.BlockSpec((1, 3, SUB, LN), lambda b: (b, 0, 0, 0))],
        out_specs=pl.BlockSpec((1, 8, N2 // 8), lambda b: (b, 0, 0)),
        out_shape=jax.ShapeDtypeStruct((B, 8, N2 // 8), jnp.int32))(pos)
    return out.reshape(B, N2)


def _pe_pre_kernel(gpos, posf, W, b, ns, bm=512):
    M = posf.shape[0]
    bm = min(bm, M)
    Wp = jnp.zeros((16, 16), jnp.float32).at[:3, :3].set(W)
    bp = jnp.zeros((16,), jnp.float32).at[:3].set(b)

    def body(g_ref, p_ref, w_ref, b_ref, o_ref, s1_ref, s2_ref):
        i = pl.program_id(0)
        g = g_ref[...].reshape(bm, ns, 16)
        p = p_ref[...]
        pr = g - p[:, None, :]
        pr = pr.reshape(bm * ns, 16)
        y = jnp.dot(pr, w_ref[...], preferred_element_type=jnp.float32) + b_ref[...]
        o_ref[...] = y
        @pl.when(i == 0)
        def _():
            s1_ref[...] = jnp.zeros_like(s1_ref)
            s2_ref[...] = jnp.zeros_like(s2_ref)
        s1_ref[...] += jnp.sum(y, 0, keepdims=True)
        s2_ref[...] += jnp.sum(y * y, 0, keepdims=True)

    outs = pl.pallas_call(
        body, grid=(M // bm,),
        in_specs=[pl.BlockSpec((bm * ns, 16), lambda i: (i, 0)),
                  pl.BlockSpec((bm, 16), lambda i: (i, 0)),
                  pl.BlockSpec((16, 16), lambda i: (0, 0)),
                  pl.BlockSpec((1, 16), lambda i: (0, 0))],
        out_specs=[pl.BlockSpec((bm * ns, 16), lambda i: (i, 0)),
                   pl.BlockSpec((1, 16), lambda i: (0, 0)),
                   pl.BlockSpec((1, 16), lambda i: (0, 0))],
        out_shape=[jax.ShapeDtypeStruct((M * ns, 16), jnp.float32),
                   jax.ShapeDtypeStruct((1, 16), jnp.float32),
                   jax.ShapeDtypeStruct((1, 16), jnp.float32)])(
        gpos, posf, Wp, _row2(bp))
    return outs[0], (outs[1][:, :3], outs[2][:, :3])


def _pe_w_kernel(pe_pre, x_k, q, s, t, W2, b2, ns, c, bm=512):
    M = q.shape[0]
    bm = min(bm, M)
    sp = jnp.zeros((16,), jnp.float32).at[:3].set(s)
    tp = jnp.zeros((16,), jnp.float32).at[:3].set(t)
    W2p = jnp.zeros((16, c), jnp.float32).at[:3, :].set(W2)

    def body(pp_ref, xk_ref, q_ref, s_ref, t_ref, w_ref, b_ref,
             pe_ref, wr_ref, s1_ref, s2_ref):
        i = pl.program_id(0)
        a = jnp.maximum(pp_ref[...] * s_ref[...] + t_ref[...], 0.0)
        pe = jnp.dot(a, w_ref[...], preferred_element_type=jnp.float32) + b_ref[...]
        pe_ref[...] = pe
        w = (xk_ref[...].reshape(bm, ns, c) - q_ref[...][:, None, :]).reshape(
            bm * ns, c) + pe
        wr_ref[...] = w
        @pl.when(i == 0)
        def _():
            s1_ref[...] = jnp.zeros_like(s1_ref)
            s2_ref[...] = jnp.zeros_like(s2_ref)
        s1_ref[...] += jnp.sum(w, 0, keepdims=True)
        s2_ref[...] += jnp.sum(w * w, 0, keepdims=True)

    outs = pl.pallas_call(
        body, grid=(M // bm,),
        in_specs=[pl.BlockSpec((bm * ns, 16), lambda i: (i, 0)),
                  pl.BlockSpec((bm * ns, c), lambda i: (i, 0)),
                  pl.BlockSpec((bm, c), lambda i: (i, 0)),
                  pl.BlockSpec((1, 16), lambda i: (0, 0)),
                  pl.BlockSpec((1, 16), lambda i: (0, 0)),
                  pl.BlockSpec((16, c), lambda i: (0, 0)),
                  pl.BlockSpec((1, c), lambda i: (0, 0))],
        out_specs=[pl.BlockSpec((bm * ns, c), lambda i: (i, 0)),
                   pl.BlockSpec((bm * ns, c), lambda i: (i, 0)),
                   pl.BlockSpec((1, c), lambda i: (0, 0)),
                   pl.BlockSpec((1, c), lambda i: (0, 0))],
        out_shape=[jax.ShapeDtypeStruct((M * ns, c), jnp.float32),
                   jax.ShapeDtypeStruct((M * ns, c), jnp.float32),
                   jax.ShapeDtypeStruct((1, c), jnp.float32),
                   jax.ShapeDtypeStruct((1, c), jnp.float32)])(
        pe_pre, x_k, q, _row2(sp), _row2(tp), W2p, _row2(b2))
    return outs[0], outs[1], (outs[2], outs[3])


def _attn_out_kernel(w1, x_v, pe, s, t, W2, b2, ns, c, bm=512):
    M = x_v.shape[0] // ns
    bm = min(bm, M)
    cs = c // SHARE

    def body(w1_ref, xv_ref, pe_ref, s_ref, t_ref, w_ref, b_ref,
             o_ref, s1_ref, s2_ref):
        i = pl.program_id(0)
        a = jnp.maximum(w1_ref[...] * s_ref[...] + t_ref[...], 0.0)
        w2 = jnp.dot(a, w_ref[...], preferred_element_type=jnp.float32) + b_ref[...]
        w3 = w2.reshape(bm, ns, cs)
        m = jnp.max(w3, axis=1, keepdims=True)
        e = jnp.exp(w3 - m)
        sm = e / jnp.sum(e, axis=1, keepdims=True)
        smf = jnp.concatenate([sm] * SHARE, axis=-1)
        xvpe = (xv_ref[...] + pe_ref[...]).reshape(bm, ns, c)
        out = jnp.sum(xvpe * smf, axis=1)
        o_ref[...] = out
        @pl.when(i == 0)
        def _():
            s1_ref[...] = jnp.zeros_like(s1_ref)
            s2_ref[...] = jnp.zeros_like(s2_ref)
        s1_ref[...] += jnp.sum(out, 0, keepdims=True)
        s2_ref[...] += jnp.sum(out * out, 0, keepdims=True)

    outs = pl.pallas_call(
        body, grid=(M // bm,),
        in_specs=[pl.BlockSpec((bm * ns, cs), lambda i: (i, 0)),
                  pl.BlockSpec((bm * ns, c), lambda i: (i, 0)),
                  pl.BlockSpec((bm * ns, c), lambda i: (i, 0)),
                  pl.BlockSpec((1, cs), lambda i: (0, 0)),
                  pl.BlockSpec((1, cs), lambda i: (0, 0)),
                  pl.BlockSpec((cs, cs), lambda i: (0, 0)),
                  pl.BlockSpec((1, cs), lambda i: (0, 0))],
        out_specs=[pl.BlockSpec((bm, c), lambda i: (i, 0)),
                   pl.BlockSpec((1, c), lambda i: (0, 0)),
                   pl.BlockSpec((1, c), lambda i: (0, 0))],
        out_shape=[jax.ShapeDtypeStruct((M, c), jnp.float32),
                   jax.ShapeDtypeStruct((1, c), jnp.float32),
                   jax.ShapeDtypeStruct((1, c), jnp.float32)])(
        w1, x_v, pe, _row2(s), _row2(t), W2, _row2(b2))
    return outs[0], (outs[1], outs[2])


def _td2_feat_kernel(gp, gx, p2f, Wrel, Wx, b, ns, bm=256):
    M = p2f.shape[0]
    bm = min(bm, M)
    dout = Wx.shape[1]
    Wr = jnp.zeros((16, dout), jnp.float32).at[:3, :].set(Wrel)

    def body(gp_ref, gx_ref, p_ref, wr_ref, wx_ref, b_ref, o_ref, s1_ref, s2_ref):
        i = pl.program_id(0)
        rel = (gp_ref[...].reshape(bm, ns, 16) - p_ref[...][:, None, :]).reshape(
            bm * ns, 16)
        y = (jnp.dot(rel, wr_ref[...], preferred_element_type=jnp.float32)
             + jnp.dot(gx_ref[...], wx_ref[...], preferred_element_type=jnp.float32)
             + b_ref[...])
        o_ref[...] = y
        @pl.when(i == 0)
        def _():
            s1_ref[...] = jnp.zeros_like(s1_ref)
            s2_ref[...] = jnp.zeros_like(s2_ref)
        s1_ref[...] += jnp.sum(y, 0, keepdims=True)
        s2_ref[...] += jnp.sum(y * y, 0, keepdims=True)

    din = gx.shape[1]
    outs = pl.pallas_call(
        body, grid=(M // bm,),
        in_specs=[pl.BlockSpec((bm * ns, 16), lambda i: (i, 0)),
                  pl.BlockSpec((bm * ns, din), lambda i: (i, 0)),
                  pl.BlockSpec((bm, 16), lambda i: (i, 0)),
                  pl.BlockSpec((16, dout), lambda i: (0, 0)),
                  pl.BlockSpec((din, dout), lambda i: (0, 0)),
                  pl.BlockSpec((1, dout), lambda i: (0, 0))],
        out_specs=[pl.BlockSpec((bm * ns, dout), lambda i: (i, 0)),
                   pl.BlockSpec((1, dout), lambda i: (0, 0)),
                   pl.BlockSpec((1, dout), lambda i: (0, 0))],
        out_shape=[jax.ShapeDtypeStruct((M * ns, dout), jnp.float32),
                   jax.ShapeDtypeStruct((1, dout), jnp.float32),
                   jax.ShapeDtypeStruct((1, dout), jnp.float32)])(
        gp, gx, p2f, Wr, Wx, _row2(b))
    return outs[0], (outs[1], outs[2])


def _td2_max_kernel(f_pre, s, t, ns, bm=256):
    Mns, d = f_pre.shape
    M = Mns // ns
    bm = min(bm, M)

    def body(f_ref, s_ref, t_ref, o_ref):
        a = jnp.maximum(f_ref[...] * s_ref[...] + t_ref[...], 0.0)
        o_ref[...] = jnp.max(a.reshape(bm, ns, d), axis=1)

    return pl.pallas_call(
        body, grid=(M // bm,),
        in_specs=[pl.BlockSpec((bm * ns, d), lambda i: (i, 0)),
                  pl.BlockSpec((1, d), lambda i: (0, 0)),
                  pl.BlockSpec((1, d), lambda i: (0, 0))],
        out_specs=pl.BlockSpec((bm, d), lambda i: (i, 0)),
        out_shape=jax.ShapeDtypeStruct((M, d), jnp.float32))(
        f_pre, _row2(s), _row2(t))


def _dec2_pre_kernel(x2b, W2, b2, W1a, W1b, b1):
    c = x2b.shape[1]

    def body(x_ref, w2_ref, b2_ref, wa_ref, wb_ref, b1_ref, o_ref, s1_ref, s2_ref):
        s1_ref[...] = jnp.zeros_like(s1_ref)
        s2_ref[...] = jnp.zeros_like(s2_ref)
        for bb in range(B):
            xb = x_ref[pl.ds(bb * N2, N2), :]
            gm = jnp.sum(xb, 0, keepdims=True) / N2
            g2 = jnp.maximum(jnp.dot(gm, w2_ref[...],
                                     preferred_element_type=jnp.float32)
                             + b2_ref[...], 0.0)
            y = (jnp.dot(xb, wa_ref[...], preferred_element_type=jnp.float32)
                 + jnp.dot(g2, wb_ref[...], preferred_element_type=jnp.float32)
                 + b1_ref[...])
            o_ref[pl.ds(bb * N2, N2), :] = y
            s1_ref[...] += jnp.sum(y, 0, keepdims=True)
            s2_ref[...] += jnp.sum(y * y, 0, keepdims=True)

    outs = pl.pallas_call(
        body,
        in_specs=[pl.BlockSpec(x2b.shape, lambda: (0, 0))] +
                 [pl.BlockSpec(a.shape, lambda: (0, 0)) for a in
                  (W2, _row2(b2), W1a, W1b, _row2(b1))],
        out_specs=[pl.BlockSpec((B * N2, c), lambda: (0, 0)),
                   pl.BlockSpec((1, c), lambda: (0, 0)),
                   pl.BlockSpec((1, c), lambda: (0, 0))],
        out_shape=[jax.ShapeDtypeStruct((B * N2, c), jnp.float32),
                   jax.ShapeDtypeStruct((1, c), jnp.float32),
                   jax.ShapeDtypeStruct((1, c), jnp.float32)])(
        x2b, W2, _row2(b2), W1a, W1b, _row2(b1))
    return outs[0], (outs[1], outs[2])


def _interp_kernel(a_pre, sa, ta, gpi, p1f, gxi, bm=1024):
    M, c = a_pre.shape
    bm = min(bm, M)

    def body(a_ref, s_ref, t_ref, gp_ref, p_ref, gx_ref, o_ref):
        a = jnp.maximum(a_ref[...] * s_ref[...] + t_ref[...], 0.0)
        gp = gp_ref[...].reshape(bm, 4, 16)[:, :3, :3]
        diff = gp - p_ref[...][:, None, :3]
        d = jnp.sqrt(jnp.sum(diff * diff, axis=-1)) + 1e-8
        w = 1.0 / d
        w = w / jnp.sum(w, -1, keepdims=True)
        gx = gx_ref[...].reshape(bm, 4, c)[:, :3, :]
        o_ref[...] = a + jnp.sum(gx * w[:, :, None], axis=1)

    return pl.pallas_call(
        body, grid=(M // bm,),
        in_specs=[pl.BlockSpec((bm, c), lambda i: (i, 0)),
                  pl.BlockSpec((1, c), lambda i: (0, 0)),
                  pl.BlockSpec((1, c), lambda i: (0, 0)),
                  pl.BlockSpec((bm * 4, 16), lambda i: (i, 0)),
                  pl.BlockSpec((bm, 16), lambda i: (i, 0)),
                  pl.BlockSpec((bm * 4, c), lambda i: (i, 0))],
        out_specs=pl.BlockSpec((bm, c), lambda i: (i, 0)),
        out_shape=jax.ShapeDtypeStruct((M, c), jnp.float32))(
        a_pre, _row2(sa), _row2(ta), gpi, p1f, gxi)


def _heads_kernel(hh_pre, s, t, Wc, bc, We, be, bm=2048):
    M = hh_pre.shape[0]
    bm = min(bm, M)

    def body(h_ref, s_ref, t_ref, wc_ref, bc_ref, we_ref, be_ref, c_ref, e_ref):
        h = jnp.maximum(h_ref[...] * s_ref[...] + t_ref[...], 0.0)
        c_ref[...] = jnp.dot(h[:, :32], wc_ref[...],
                             preferred_element_type=jnp.float32) + bc_ref[...]
        e_ref[...] = jnp.dot(h[:, 32:], we_ref[...],
                             preferred_element_type=jnp.float32) + be_ref[...]

    return pl.pallas_call(
        body, grid=(M // bm,),
        in_specs=[pl.BlockSpec((bm, 64), lambda i: (i, 0)),
                  pl.BlockSpec((1, 64), lambda i: (0, 0)),
                  pl.BlockSpec((1, 64), lambda i: (0, 0)),
                  pl.BlockSpec((32, K_CLS), lambda i: (0, 0)),
                  pl.BlockSpec((1, K_CLS), lambda i: (0, 0)),
                  pl.BlockSpec((32, 2), lambda i: (0, 0)),
                  pl.BlockSpec((1, 2), lambda i: (0, 0))],
        out_specs=[pl.BlockSpec((bm, K_CLS), lambda i: (i, 0)),
                   pl.BlockSpec((bm, 2), lambda i: (i, 0))],
        out_shape=[jax.ShapeDtypeStruct((M, K_CLS), jnp.float32),
                   jax.ShapeDtypeStruct((M, 2), jnp.float32)])(
        hh_pre, _row2(s), _row2(t), Wc, _row2(bc), We, _row2(be))


def _pt_layer(p, posf16, gpos16, q, kf, v, idx, ns, c):
    M = q.shape[0]
    pe_pre, st = _pe_pre_kernel(gpos16, posf16, p["p1"]["W"], p["p1"]["b"], ns)
    s, t = _bn_scale_shift(p["pbn"], st[0], st[1], M * ns)

    kv = jnp.concatenate([kf, v], axis=1)
    gkv = _gather_rows(kv, idx)
    x_k = gkv[:, :c]
    x_v = gkv[:, c:]

    pe, w_raw, st = _pe_w_kernel(pe_pre, x_k, q, s, t,
                                 p["p2"]["W"], p["p2"]["b"], ns, c)
    s, t = _bn_scale_shift(p["wbn1"], st[0], st[1], M * ns)
    w1, st = _dense(w_raw, p["w1"]["W"], p["w1"]["b"], pre=(s, t))
    s, t = _bn_scale_shift(p["wbn2"], st[0], st[1], M * ns)
    out, st = _attn_out_kernel(w1, x_v, pe, s, t,
                               p["w2"]["W"], p["w2"]["b"], ns, c)
    return out, st


def _pt_block(p, posf16, gpos16, x, idx, ns, c):
    h_pre, st = _dense(x, p["l1"]["W"], p["l1"]["b"])
    return _pt_block_from(p, posf16, gpos16, x, h_pre, st, idx, ns, c)


def _pad16(a):
    return jnp.concatenate([a, jnp.zeros(a.shape[:-1] + (16 - a.shape[-1],),
                                         jnp.float32)], -1)


def _forward(inputs, params):
    pxo = jnp.transpose(inputs, (0, 2, 1))
    x0 = pxo.reshape(B * N, C)
    p1 = pxo[:, :, :3]
    p1f = p1.reshape(B * N, 3)
    p1f16 = _pad16(p1f)

    td = params["enc1_td"]
    y_pre, st = _dense(x0, td["lin"]["W"], td["lin"]["b"])
    s, t = _bn_scale_shift(td["bn"], st[0], st[1], B * N)

    idx1 = _knn_pallas(p1, p1, NSAMPLE[0])
    gidx1 = (idx1 + (jnp.arange(B, dtype=jnp.int32) * N)[:, None, None]).reshape(-1)
    gpos1 = _gather_rows(p1f16, gidx1)

    blk = params["enc1_blk"]
    h_pre, x1, st1 = _dense(y_pre, blk["l1"]["W"], blk["l1"]["b"],
                            pre=(s, t), emit_a=True)
    x1b = _pt_block_from(blk, p1f16, gpos1, x1, h_pre, st1, gidx1,
                         NSAMPLE[0], PLANES[0])

    sidx = _fps_pallas(p1)
    gsidx = (sidx + (jnp.arange(B, dtype=jnp.int32) * N)[:, None]).reshape(-1)
    p2f16 = _gather_rows(p1f16, gsidx)
    p2f = p2f16[:, :3]
    p2 = p2f.reshape(B, N2, 3)
    nidx = _knn_pallas(p2, p1, NSAMPLE[1])
    gnidx = (nidx + (jnp.arange(B, dtype=jnp.int32) * N)[:, None, None]).reshape(-1)
    gp = _gather_rows(p1f16, gnidx)
    gx = _gather_rows(x1b, gnidx)
    td = params["enc2_td"]
    f_pre, st = _td2_feat_kernel(gp, gx, p2f16, td["lin"]["W"][:3],
                                 td["lin"]["W"][3:], td["lin"]["b"], NSAMPLE[1])
    s, t = _bn_scale_shift(td["bn"], st[0], st[1], B * N2 * NSAMPLE[1])
    x2 = _td2_max_kernel(f_pre, s, t, NSAMPLE[1])

    idx2 = _knn_pallas(p2, p2, NSAMPLE[1])
    gidx2 = (idx2 + (jnp.arange(B, dtype=jnp.int32) * N2)[:, None, None]).reshape(-1)
    gpos2 = _gather_rows(p2f16, gidx2)

    x2b = _pt_block(params["enc2_blk"], p2f16, gpos2, x2, gidx2,
                    NSAMPLE[1], PLANES[1])

    up = params["dec2_up"]
    h2_pre, st = _dec2_pre_kernel(x2b, up["l2"]["W"], up["l2"]["b"],
                                  up["l1"]["W"][:PLANES[1]],
                                  up["l1"]["W"][PLANES[1]:], up["l1"]["b"])
    s, t = _bn_scale_shift(up["bn1"], st[0], st[1], B * N2)
    h2 = _ew_bnrelu(h2_pre, s, t)
    x2d = _pt_block(params["dec2_blk"], p2f16, gpos2, h2, gidx2,
                    NSAMPLE[1], PLANES[1])

    up = params["dec1_up"]
    a_pre, sta = _dense(x1b, up["l1"]["W"], up["l1"]["b"])
    sa, ta = _bn_scale_shift(up["bn1"], sta[0], sta[1], B * N)
    b_pre, stb = _dense(x2d, up["l2"]["W"], up["l2"]["b"])
    sb, tb = _bn_scale_shift(up["bn2"], stb[0], stb[1], B * N2)
    bfeat = _ew_bnrelu(b_pre, sb, tb)

    iidx = _knn_pallas(p1, p2, 3)
    iidx4 = jnp.concatenate([iidx, iidx[:, :, :1]], axis=-1)
    giidx = (iidx4 + (jnp.arange(B, dtype=jnp.int32) * N2)[:, None, None]
             ).reshape(-1)
    gpi = _gather_rows(p2f16, giidx)
    gxi = _gather_rows(bfeat, giidx)
    h1 = _interp_kernel(a_pre, sa, ta, gpi, p1f16, gxi)

    x1d = _pt_block(params["dec1_blk"], p1f16, gpos1, h1, gidx1,
                    NSAMPLE[0], PLANES[0])

    ch, eh = params["cls_head"], params["edge_head"]
    Wcat = jnp.concatenate([ch["l1"]["W"], eh["l1"]["W"]], axis=1)
    bcat = jnp.concatenate([ch["l1"]["b"], eh["l1"]["b"]])
    hh_pre, st = _dense(x1d, Wcat, bcat)
    sA, tA = _bn_scale_shift(ch["bn"], st[0][:, :32], st[1][:, :32], B * N)
    sB, tB = _bn_scale_shift(eh["bn"], st[0][:, 32:], st[1][:, 32:], B * N)
    s = jnp.concatenate([sA, sB])
    t = jnp.concatenate([tA, tB])
    cls, edge = _heads_kernel(hh_pre, s, t, ch["l2"]["W"], ch["l2"]["b"],
                              eh["l2"]["W"], eh["l2"]["b"])
    cls = jnp.transpose(cls.reshape(B, N, K_CLS), (0, 2, 1))
    edge = jnp.transpose(edge.reshape(B, N, 2), (0, 2, 1))
    return (cls, edge)


def _pt_block_from(p, posf16, gpos16, x, h_pre, st, idx, ns, c):
    M = x.shape[0]
    s, t = _bn_scale_shift(p["bn1"], st[0], st[1], M)
    tr = p["tr"]
    Wqkv = jnp.concatenate([tr["q"]["W"], tr["k"]["W"], tr["v"]["W"]], axis=1)
    bqkv = jnp.concatenate([tr["q"]["b"], tr["k"]["b"], tr["v"]["b"]])
    (qkv,) = _dense(h_pre, Wqkv, bqkv, pre=(s, t), want_stats=False)
    q, kf, v = qkv[:, :c], qkv[:, c:2 * c], qkv[:, 2 * c:]
    tt, st = _pt_layer(tr, posf16, gpos16, q, kf, v, idx, ns, c)
    s, t = _bn_scale_shift(p["bn2"], st[0], st[1], M)
    h3_pre, st = _dense(tt, p["l3"]["W"], p["l3"]["b"], pre=(s, t))
    s, t = _bn_scale_shift(p["bn3"], st[0], st[1], M)
    return _ew_bnrelu(h3_pre, s, t, add=x)


def kernel(inputs, params):
    return _forward(inputs, params)


# speedup vs baseline: 8.8386x; 1.0509x over previous
import functools
import jax
import jax.numpy as jnp
import numpy as np
from jax.experimental import pallas as pl
from jax.experimental.pallas import tpu as pltpu
from jax.experimental.pallas import tpu_sc as plsc

B, C, N = 2, 6, 4096
K_CLS = 13
PLANES = [32, 64]
NSAMPLE = [8, 16]
SHARE = 8
N2 = N // 4
EPS = 1e-5
NEG_BIG = 3.0e38


def _bn_scale_shift(p, s1, s2, count):
    s1 = s1.reshape(-1)
    s2 = s2.reshape(-1)
    m = s1 / count
    v = s2 / count - m * m
    s = p["g"] * jax.lax.rsqrt(v + EPS)
    t = p["b"] - m * s
    return s, t


def _row2(a):
    return a.reshape(1, -1)


def _gather_rows(table, idx):
    R, D = table.shape
    Rout = idx.shape[0]
    NW = 32
    assert Rout % NW == 0 and D % 16 == 0
    per_w = Rout // NW
    ch = min(128, per_w)
    assert per_w % ch == 0
    n_chunks = per_w // ch

    mesh = plsc.VectorSubcoreMesh(core_axis_name="c", subcore_axis_name="s")

    @functools.partial(
        pl.kernel, mesh=mesh,
        out_type=jax.ShapeDtypeStruct((Rout, D), jnp.float32),
        compiler_params=pltpu.CompilerParams(use_tc_tiling_on_sc=False),
        scratch_types=[
            pltpu.VMEM((per_w,), jnp.int32),
            pltpu.VMEM((ch, D), jnp.float32),
            pltpu.VMEM((ch, D), jnp.float32),
            pltpu.SemaphoreType.DMA,
            pltpu.SemaphoreType.DMA,
        ],
    )
    def k(table_hbm, idx_hbm, out_hbm, idx_v, rows0, rows1, sem0, sem1):
        wid = jax.lax.axis_index("s") * 2 + jax.lax.axis_index("c")
        base = wid * per_w
        pltpu.sync_copy(idx_hbm.at[pl.ds(base, per_w)], idx_v)
        bufs = [(rows0, sem0), (rows1, sem1)]
        cps = [None, None]
        for ci in range(n_chunks):
            rv, sem = bufs[ci % 2]
            cps[ci % 2] = pltpu.async_copy(
                table_hbm.at[idx_v.at[pl.ds(ci * ch, ch)]], rv, sem)
            if ci > 0:
                pv, psem = bufs[(ci - 1) % 2]
                cps[(ci - 1) % 2].wait()
                pltpu.sync_copy(pv, out_hbm.at[pl.ds(base + (ci - 1) * ch, ch)])
        lv, lsem = bufs[(n_chunks - 1) % 2]
        cps[(n_chunks - 1) % 2].wait()
        pltpu.sync_copy(lv, out_hbm.at[pl.ds(base + (n_chunks - 1) * ch, ch)])

    return k(table, idx)


def _dense(X, W, b, pre=None, want_stats=True, emit_a=False, bm=2048):
    M, din = X.shape
    dout = W.shape[1]
    bm = min(bm, M)
    grid = M // bm
    assert M % bm == 0

    def body(*refs):
        i = pl.program_id(0)
        ir = iter(refs)
        x_ref = next(ir)
        w_ref = next(ir)
        b_ref = next(ir)
        if pre is not None:
            s_ref = next(ir)
            t_ref = next(ir)
        y_ref = next(ir)
        a_ref = next(ir) if emit_a else None
        if want_stats:
            s1_ref = next(ir)
            s2_ref = next(ir)
        x = x_ref[...]
        if pre is not None:
            x = jnp.maximum(x * s_ref[...] + t_ref[...], 0.0)
        if emit_a:
            a_ref[...] = x
        y = jnp.dot(x, w_ref[...], preferred_element_type=jnp.float32) + b_ref[...]
        y_ref[...] = y
        if want_stats:
            @pl.when(i == 0)
            def _():
                s1_ref[...] = jnp.zeros_like(s1_ref)
                s2_ref[...] = jnp.zeros_like(s2_ref)
            s1_ref[...] += jnp.sum(y, 0, keepdims=True)
            s2_ref[...] += jnp.sum(y * y, 0, keepdims=True)

    in_specs = [
        pl.BlockSpec((bm, din), lambda i: (i, 0)),
        pl.BlockSpec((din, dout), lambda i: (0, 0)),
        pl.BlockSpec((1, dout), lambda i: (0, 0)),
    ]
    args = [X, W, _row2(b)]
    if pre is not None:
        in_specs += [pl.BlockSpec((1, din), lambda i: (0, 0))] * 2
        args += [_row2(pre[0]), _row2(pre[1])]
    out_specs = [pl.BlockSpec((bm, dout), lambda i: (i, 0))]
    out_shapes = [jax.ShapeDtypeStruct((M, dout), jnp.float32)]
    if emit_a:
        out_specs.append(pl.BlockSpec((bm, din), lambda i: (i, 0)))
        out_shapes.append(jax.ShapeDtypeStruct((M, din), jnp.float32))
    if want_stats:
        out_specs += [pl.BlockSpec((1, dout), lambda i: (0, 0))] * 2
        out_shapes += [jax.ShapeDtypeStruct((1, dout), jnp.float32)] * 2
    outs = pl.pallas_call(
        body, grid=(grid,), in_specs=in_specs, out_specs=out_specs,
        out_shape=out_shapes)(*args)
    outs = list(outs)
    y = outs.pop(0)
    a = outs.pop(0) if emit_a else None
    st = (outs[0], outs[1]) if want_stats else None
    res = [y]
    if emit_a:
        res.append(a)
    if want_stats:
        res.append(st)
    return res


def _ew_bnrelu(X, s, t, add=None, bm=2048):
    M, d = X.shape
    bm = min(bm, M)

    def body(*refs):
        if add is not None:
            x_ref, s_ref, t_ref, a_ref, o_ref = refs
        else:
            x_ref, s_ref, t_ref, o_ref = refs
        y = x_ref[...] * s_ref[...] + t_ref[...]
        if add is not None:
            y = y + a_ref[...]
        o_ref[...] = jnp.maximum(y, 0.0)

    in_specs = [pl.BlockSpec((bm, d), lambda i: (i, 0)),
                pl.BlockSpec((1, d), lambda i: (0, 0)),
                pl.BlockSpec((1, d), lambda i: (0, 0))]
    args = [X, _row2(s), _row2(t)]
    if add is not None:
        in_specs.append(pl.BlockSpec((bm, d), lambda i: (i, 0)))
        args.append(add)
    return pl.pallas_call(
        body, grid=(M // bm,), in_specs=in_specs,
        out_specs=pl.BlockSpec((bm, d), lambda i: (i, 0)),
        out_shape=jax.ShapeDtypeStruct((M, d), jnp.float32))(*args)


def _knn_pallas(qpos, rpos, k, bq=256):
    Bq, Mq, _ = qpos.shape
    Mr = rpos.shape[1]
    qp = jnp.concatenate([qpos, jnp.zeros((Bq, Mq, 13), jnp.float32)], -1)
    rt = jnp.transpose(rpos, (0, 2, 1))
    rt = jnp.concatenate([rt, jnp.zeros((Bq, 5, Mr), jnp.float32)], 1)

    def body(q_ref, r_ref, idx_ref):
        q = q_ref[0]
        r = r_ref[0]
        q3 = q[:, :3]
        r3 = r[:3, :]
        qq = jnp.sum(q3 * q3, 1, keepdims=True)
        rr = jnp.sum(r3 * r3, 0, keepdims=True)
        cross = jnp.dot(q3, r3, preferred_element_type=jnp.float32)
        d = qq - 2.0 * cross + rr
        iota_r = jax.lax.broadcasted_iota(jnp.int32, (bq, Mr), 1)
        for j in range(k):
            m = jnp.min(d, 1, keepdims=True)
            am = jnp.min(jnp.where(d == m, iota_r, Mr), 1, keepdims=True)
            idx_ref[0, :, pl.ds(j, 1)] = am
            d = jnp.where(iota_r == am, NEG_BIG, d)

    return pl.pallas_call(
        body, grid=(Bq, Mq // bq),
        in_specs=[pl.BlockSpec((1, bq, 16), lambda b, i: (b, i, 0)),
                  pl.BlockSpec((1, 8, Mr), lambda b, i: (b, 0, 0))],
        out_specs=pl.BlockSpec((1, bq, k), lambda b, i: (b, i, 0)),
        out_shape=jax.ShapeDtypeStruct((Bq, Mq, k), jnp.int32))(qp, rt)


def _fps_pallas(p1):
    SUB, LN = 8, N // 8
    pos = jnp.transpose(p1, (0, 2, 1)).reshape(B, 3, SUB, LN)
    posrow = jnp.concatenate(
        [p1, jnp.zeros((B, N, 5), jnp.float32)], -1)

    def body(p_ref, pr_ref, o_ref):
        ii = (jax.lax.broadcasted_iota(jnp.int32, (SUB, LN), 0) * LN
              + jax.lax.broadcasted_iota(jnp.int32, (SUB, LN), 1))
        ii_out = (jax.lax.broadcasted_iota(jnp.int32, (8, N2 // 8), 0) * (N2 // 8)
                  + jax.lax.broadcasted_iota(jnp.int32, (8, N2 // 8), 1))
        coords = [(p_ref[b, 0], p_ref[b, 1], p_ref[b, 2]) for b in range(B)]

        def step(i, st):
            dists, idxs, lasts = st
            new_dists, new_idxs, new_lasts = [], [], []
            for b in range(B):
                xr, yr, zr = coords[b]
                row = pr_ref[b, pl.ds(lasts[b], 1), :]
                lx = row[:, 0:1]
                ly = row[:, 1:2]
                lz = row[:, 2:3]
                d = (xr - lx) ** 2 + (yr - ly) ** 2 + (zr - lz) ** 2
                dist = jnp.minimum(dists[b], d)
                m = jnp.max(dist)
                g = jnp.min(jnp.where(dist == m, ii, N))
                new_dists.append(dist)
                new_idxs.append(jnp.where(ii_out == i, g, idxs[b]))
                new_lasts.append(g)
            return tuple(new_dists), tuple(new_idxs), tuple(new_lasts)

        dist0 = tuple(jnp.full((SUB, LN), 1e10, jnp.float32) for _ in range(B))
        idx0 = tuple(jnp.zeros((8, N2 // 8), jnp.int32) for _ in range(B))
        _, idx_arr, _ = jax.lax.fori_loop(
            1, N2, step, (dist0, idx0, tuple(0 for _ in range(B))))
        for b in range(B):
            o_ref[b] = idx_arr[b]

    out = pl.pallas_call(
        body,
        in_specs=[pl.BlockSpec(pos.shape, lambda: (0, 0, 0, 0)),
                  pl.BlockSpec(posrow.shape, lambda: (0, 0, 0))],
        out_specs=pl.BlockSpec((B, 8, N2 // 8), lambda: (0, 0, 0)),
        out_shape=jax.ShapeDtypeStruct((B, 8, N2 // 8), jnp.int32))(pos, posrow)
    return out.reshape(B, N2)


def _pe_pre_kernel(gpos, posf, W, b, ns, bm=512):
    M = posf.shape[0]
    bm = min(bm, M)
    Wp = jnp.zeros((16, 16), jnp.float32).at[:3, :3].set(W)
    bp = jnp.zeros((16,), jnp.float32).at[:3].set(b)

    def body(g_ref, p_ref, w_ref, b_ref, o_ref, s1_ref, s2_ref):
        i = pl.program_id(0)
        g = g_ref[...].reshape(bm, ns, 16)
        p = p_ref[...]
        pr = g - p[:, None, :]
        pr = pr.reshape(bm * ns, 16)
        y = jnp.dot(pr, w_ref[...], preferred_element_type=jnp.float32) + b_ref[...]
        o_ref[...] = y
        @pl.when(i == 0)
        def _():
            s1_ref[...] = jnp.zeros_like(s1_ref)
            s2_ref[...] = jnp.zeros_like(s2_ref)
        s1_ref[...] += jnp.sum(y, 0, keepdims=True)
        s2_ref[...] += jnp.sum(y * y, 0, keepdims=True)

    outs = pl.pallas_call(
        body, grid=(M // bm,),
        in_specs=[pl.BlockSpec((bm * ns, 16), lambda i: (i, 0)),
                  pl.BlockSpec((bm, 16), lambda i: (i, 0)),
                  pl.BlockSpec((16, 16), lambda i: (0, 0)),
                  pl.BlockSpec((1, 16), lambda i: (0, 0))],
        out_specs=[pl.BlockSpec((bm * ns, 16), lambda i: (i, 0)),
                   pl.BlockSpec((1, 16), lambda i: (0, 0)),
                   pl.BlockSpec((1, 16), lambda i: (0, 0))],
        out_shape=[jax.ShapeDtypeStruct((M * ns, 16), jnp.float32),
                   jax.ShapeDtypeStruct((1, 16), jnp.float32),
                   jax.ShapeDtypeStruct((1, 16), jnp.float32)])(
        gpos, posf, Wp, _row2(bp))
    return outs[0], (outs[1][:, :3], outs[2][:, :3])


def _pe_w_kernel(pe_pre, x_k, q, s, t, W2, b2, ns, c, bm=512):
    M = q.shape[0]
    bm = min(bm, M)
    sp = jnp.zeros((16,), jnp.float32).at[:3].set(s)
    tp = jnp.zeros((16,), jnp.float32).at[:3].set(t)
    W2p = jnp.zeros((16, c), jnp.float32).at[:3, :].set(W2)

    def body(pp_ref, xk_ref, q_ref, s_ref, t_ref, w_ref, b_ref,
             pe_ref, wr_ref, s1_ref, s2_ref):
        i = pl.program_id(0)
        a = jnp.maximum(pp_ref[...] * s_ref[...] + t_ref[...], 0.0)
        pe = jnp.dot(a, w_ref[...], preferred_element_type=jnp.float32) + b_ref[...]
        pe_ref[...] = pe
        w = (xk_ref[...].reshape(bm, ns, c) - q_ref[...][:, None, :]).reshape(
            bm * ns, c) + pe
        wr_ref[...] = w
        @pl.when(i == 0)
        def _():
            s1_ref[...] = jnp.zeros_like(s1_ref)
            s2_ref[...] = jnp.zeros_like(s2_ref)
        s1_ref[...] += jnp.sum(w, 0, keepdims=True)
        s2_ref[...] += jnp.sum(w * w, 0, keepdims=True)

    outs = pl.pallas_call(
        body, grid=(M // bm,),
        in_specs=[pl.BlockSpec((bm * ns, 16), lambda i: (i, 0)),
                  pl.BlockSpec((bm * ns, c), lambda i: (i, 0)),
                  pl.BlockSpec((bm, c), lambda i: (i, 0)),
                  pl.BlockSpec((1, 16), lambda i: (0, 0)),
                  pl.BlockSpec((1, 16), lambda i: (0, 0)),
                  pl.BlockSpec((16, c), lambda i: (0, 0)),
                  pl.BlockSpec((1, c), lambda i: (0, 0))],
        out_specs=[pl.BlockSpec((bm * ns, c), lambda i: (i, 0)),
                   pl.BlockSpec((bm * ns, c), lambda i: (i, 0)),
                   pl.BlockSpec((1, c), lambda i: (0, 0)),
                   pl.BlockSpec((1, c), lambda i: (0, 0))],
        out_shape=[jax.ShapeDtypeStruct((M * ns, c), jnp.float32),
                   jax.ShapeDtypeStruct((M * ns, c), jnp.float32),
                   jax.ShapeDtypeStruct((1, c), jnp.float32),
                   jax.ShapeDtypeStruct((1, c), jnp.float32)])(
        pe_pre, x_k, q, _row2(sp), _row2(tp), W2p, _row2(b2))
    return outs[0], outs[1], (outs[2], outs[3])


def _attn_out_kernel(w1, x_v, pe, s, t, W2, b2, ns, c, bm=512):
    M = x_v.shape[0] // ns
    bm = min(bm, M)
    cs = c // SHARE

    def body(w1_ref, xv_ref, pe_ref, s_ref, t_ref, w_ref, b_ref,
             o_ref, s1_ref, s2_ref):
        i = pl.program_id(0)
        a = jnp.maximum(w1_ref[...] * s_ref[...] + t_ref[...], 0.0)
        w2 = jnp.dot(a, w_ref[...], preferred_element_type=jnp.float32) + b_ref[...]
        w3 = w2.reshape(bm, ns, cs)
        m = jnp.max(w3, axis=1, keepdims=True)
        e = jnp.exp(w3 - m)
        sm = e / jnp.sum(e, axis=1, keepdims=True)
        smf = jnp.concatenate([sm] * SHARE, axis=-1)
        xvpe = (xv_ref[...] + pe_ref[...]).reshape(bm, ns, c)
        out = jnp.sum(xvpe * smf, axis=1)
        o_ref[...] = out
        @pl.when(i == 0)
        def _():
            s1_ref[...] = jnp.zeros_like(s1_ref)
            s2_ref[...] = jnp.zeros_like(s2_ref)
        s1_ref[...] += jnp.sum(out, 0, keepdims=True)
        s2_ref[...] += jnp.sum(out * out, 0, keepdims=True)

    outs = pl.pallas_call(
        body, grid=(M // bm,),
        in_specs=[pl.BlockSpec((bm * ns, cs), lambda i: (i, 0)),
                  pl.BlockSpec((bm * ns, c), lambda i: (i, 0)),
                  pl.BlockSpec((bm * ns, c), lambda i: (i, 0)),
                  pl.BlockSpec((1, cs), lambda i: (0, 0)),
                  pl.BlockSpec((1, cs), lambda i: (0, 0)),
                  pl.BlockSpec((cs, cs), lambda i: (0, 0)),
                  pl.BlockSpec((1, cs), lambda i: (0, 0))],
        out_specs=[pl.BlockSpec((bm, c), lambda i: (i, 0)),
                   pl.BlockSpec((1, c), lambda i: (0, 0)),
                   pl.BlockSpec((1, c), lambda i: (0, 0))],
        out_shape=[jax.ShapeDtypeStruct((M, c), jnp.float32),
                   jax.ShapeDtypeStruct((1, c), jnp.float32),
                   jax.ShapeDtypeStruct((1, c), jnp.float32)])(
        w1, x_v, pe, _row2(s), _row2(t), W2, _row2(b2))
    return outs[0], (outs[1], outs[2])


def _td2_feat_kernel(gp, gx, p2f, Wrel, Wx, b, ns, bm=256):
    M = p2f.shape[0]
    bm = min(bm, M)
    dout = Wx.shape[1]
    Wr = jnp.zeros((16, dout), jnp.float32).at[:3, :].set(Wrel)

    def body(gp_ref, gx_ref, p_ref, wr_ref, wx_ref, b_ref, o_ref, s1_ref, s2_ref):
        i = pl.program_id(0)
        rel = (gp_ref[...].reshape(bm, ns, 16) - p_ref[...][:, None, :]).reshape(
            bm * ns, 16)
        y = (jnp.dot(rel, wr_ref[...], preferred_element_type=jnp.float32)
             + jnp.dot(gx_ref[...], wx_ref[...], preferred_element_type=jnp.float32)
             + b_ref[...])
        o_ref[...] = y
        @pl.when(i == 0)
        def _():
            s1_ref[...] = jnp.zeros_like(s1_ref)
            s2_ref[...] = jnp.zeros_like(s2_ref)
        s1_ref[...] += jnp.sum(y, 0, keepdims=True)
        s2_ref[...] += jnp.sum(y * y, 0, keepdims=True)

    din = gx.shape[1]
    outs = pl.pallas_call(
        body, grid=(M // bm,),
        in_specs=[pl.BlockSpec((bm * ns, 16), lambda i: (i, 0)),
                  pl.BlockSpec((bm * ns, din), lambda i: (i, 0)),
                  pl.BlockSpec((bm, 16), lambda i: (i, 0)),
                  pl.BlockSpec((16, dout), lambda i: (0, 0)),
                  pl.BlockSpec((din, dout), lambda i: (0, 0)),
                  pl.BlockSpec((1, dout), lambda i: (0, 0))],
        out_specs=[pl.BlockSpec((bm * ns, dout), lambda i: (i, 0)),
                   pl.BlockSpec((1, dout), lambda i: (0, 0)),
                   pl.BlockSpec((1, dout), lambda i: (0, 0))],
        out_shape=[jax.ShapeDtypeStruct((M * ns, dout), jnp.float32),
                   jax.ShapeDtypeStruct((1, dout), jnp.float32),
                   jax.ShapeDtypeStruct((1, dout), jnp.float32)])(
        gp, gx, p2f, Wr, Wx, _row2(b))
    return outs[0], (outs[1], outs[2])


def _td2_max_kernel(f_pre, s, t, ns, bm=256):
    Mns, d = f_pre.shape
    M = Mns // ns
    bm = min(bm, M)

    def body(f_ref, s_ref, t_ref, o_ref):
        a = jnp.maximum(f_ref[...] * s_ref[...] + t_ref[...], 0.0)
        o_ref[...] = jnp.max(a.reshape(bm, ns, d), axis=1)

    return pl.pallas_call(
        body, grid=(M // bm,),
        in_specs=[pl.BlockSpec((bm * ns, d), lambda i: (i, 0)),
                  pl.BlockSpec((1, d), lambda i: (0, 0)),
                  pl.BlockSpec((1, d), lambda i: (0, 0))],
        out_specs=pl.BlockSpec((bm, d), lambda i: (i, 0)),
        out_shape=jax.ShapeDtypeStruct((M, d), jnp.float32))(
        f_pre, _row2(s), _row2(t))


def _dec2_pre_kernel(x2b, W2, b2, W1a, W1b, b1):
    c = x2b.shape[1]

    def body(x_ref, w2_ref, b2_ref, wa_ref, wb_ref, b1_ref, o_ref, s1_ref, s2_ref):
        s1_ref[...] = jnp.zeros_like(s1_ref)
        s2_ref[...] = jnp.zeros_like(s2_ref)
        for bb in range(B):
            xb = x_ref[pl.ds(bb * N2, N2), :]
            gm = jnp.sum(xb, 0, keepdims=True) / N2
            g2 = jnp.maximum(jnp.dot(gm, w2_ref[...],
                                     preferred_element_type=jnp.float32)
                             + b2_ref[...], 0.0)
            y = (jnp.dot(xb, wa_ref[...], preferred_element_type=jnp.float32)
                 + jnp.dot(g2, wb_ref[...], preferred_element_type=jnp.float32)
                 + b1_ref[...])
            o_ref[pl.ds(bb * N2, N2), :] = y
            s1_ref[...] += jnp.sum(y, 0, keepdims=True)
            s2_ref[...] += jnp.sum(y * y, 0, keepdims=True)

    outs = pl.pallas_call(
        body,
        in_specs=[pl.BlockSpec(x2b.shape, lambda: (0, 0))] +
                 [pl.BlockSpec(a.shape, lambda: (0, 0)) for a in
                  (W2, _row2(b2), W1a, W1b, _row2(b1))],
        out_specs=[pl.BlockSpec((B * N2, c), lambda: (0, 0)),
                   pl.BlockSpec((1, c), lambda: (0, 0)),
                   pl.BlockSpec((1, c), lambda: (0, 0))],
        out_shape=[jax.ShapeDtypeStruct((B * N2, c), jnp.float32),
                   jax.ShapeDtypeStruct((1, c), jnp.float32),
                   jax.ShapeDtypeStruct((1, c), jnp.float32)])(
        x2b, W2, _row2(b2), W1a, W1b, _row2(b1))
    return outs[0], (outs[1], outs[2])


def _interp_kernel(a_pre, sa, ta, gpi, p1f, gxi, bm=1024):
    M, c = a_pre.shape
    bm = min(bm, M)

    def body(a_ref, s_ref, t_ref, gp_ref, p_ref, gx_ref, o_ref):
        a = jnp.maximum(a_ref[...] * s_ref[...] + t_ref[...], 0.0)
        gp = gp_ref[...].reshape(bm, 4, 16)[:, :3, :3]
        diff = gp - p_ref[...][:, None, :3]
        d = jnp.sqrt(jnp.sum(diff * diff, axis=-1)) + 1e-8
        w = 1.0 / d
        w = w / jnp.sum(w, -1, keepdims=True)
        gx = gx_ref[...].reshape(bm, 4, c)[:, :3, :]
        o_ref[...] = a + jnp.sum(gx * w[:, :, None], axis=1)

    return pl.pallas_call(
        body, grid=(M // bm,),
        in_specs=[pl.BlockSpec((bm, c), lambda i: (i, 0)),
                  pl.BlockSpec((1, c), lambda i: (0, 0)),
                  pl.BlockSpec((1, c), lambda i: (0, 0)),
                  pl.BlockSpec((bm * 4, 16), lambda i: (i, 0)),
                  pl.BlockSpec((bm, 16), lambda i: (i, 0)),
                  pl.BlockSpec((bm * 4, c), lambda i: (i, 0))],
        out_specs=pl.BlockSpec((bm, c), lambda i: (i, 0)),
        out_shape=jax.ShapeDtypeStruct((M, c), jnp.float32))(
        a_pre, _row2(sa), _row2(ta), gpi, p1f, gxi)


def _heads_kernel(hh_pre, s, t, Wc, bc, We, be, bm=2048):
    M = hh_pre.shape[0]
    bm = min(bm, M)

    def body(h_ref, s_ref, t_ref, wc_ref, bc_ref, we_ref, be_ref, c_ref, e_ref):
        h = jnp.maximum(h_ref[...] * s_ref[...] + t_ref[...], 0.0)
        c_ref[...] = jnp.dot(h[:, :32], wc_ref[...],
                             preferred_element_type=jnp.float32) + bc_ref[...]
        e_ref[...] = jnp.dot(h[:, 32:], we_ref[...],
                             preferred_element_type=jnp.float32) + be_ref[...]

    return pl.pallas_call(
        body, grid=(M // bm,),
        in_specs=[pl.BlockSpec((bm, 64), lambda i: (i, 0)),
                  pl.BlockSpec((1, 64), lambda i: (0, 0)),
                  pl.BlockSpec((1, 64), lambda i: (0, 0)),
                  pl.BlockSpec((32, K_CLS), lambda i: (0, 0)),
                  pl.BlockSpec((1, K_CLS), lambda i: (0, 0)),
                  pl.BlockSpec((32, 2), lambda i: (0, 0)),
                  pl.BlockSpec((1, 2), lambda i: (0, 0))],
        out_specs=[pl.BlockSpec((bm, K_CLS), lambda i: (i, 0)),
                   pl.BlockSpec((bm, 2), lambda i: (i, 0))],
        out_shape=[jax.ShapeDtypeStruct((M, K_CLS), jnp.float32),
                   jax.ShapeDtypeStruct((M, 2), jnp.float32)])(
        hh_pre, _row2(s), _row2(t), Wc, _row2(bc), We, _row2(be))


def _pt_layer(p, posf16, gpos16, q, kf, v, idx, ns, c):
    M = q.shape[0]
    pe_pre, st = _pe_pre_kernel(gpos16, posf16, p["p1"]["W"], p["p1"]["b"], ns)
    s, t = _bn_scale_shift(p["pbn"], st[0], st[1], M * ns)

    kv = jnp.concatenate([kf, v], axis=1)
    gkv = _gather_rows(kv, idx)
    x_k = gkv[:, :c]
    x_v = gkv[:, c:]

    pe, w_raw, st = _pe_w_kernel(pe_pre, x_k, q, s, t,
                                 p["p2"]["W"], p["p2"]["b"], ns, c)
    s, t = _bn_scale_shift(p["wbn1"], st[0], st[1], M * ns)
    w1, st = _dense(w_raw, p["w1"]["W"], p["w1"]["b"], pre=(s, t))
    s, t = _bn_scale_shift(p["wbn2"], st[0], st[1], M * ns)
    out, st = _attn_out_kernel(w1, x_v, pe, s, t,
                               p["w2"]["W"], p["w2"]["b"], ns, c)
    return out, st


def _pt_block(p, posf16, gpos16, x, idx, ns, c):
    h_pre, st = _dense(x, p["l1"]["W"], p["l1"]["b"])
    return _pt_block_from(p, posf16, gpos16, x, h_pre, st, idx, ns, c)


def _pad16(a):
    return jnp.concatenate([a, jnp.zeros(a.shape[:-1] + (16 - a.shape[-1],),
                                         jnp.float32)], -1)


def _forward(inputs, params):
    pxo = jnp.transpose(inputs, (0, 2, 1))
    x0 = pxo.reshape(B * N, C)
    p1 = pxo[:, :, :3]
    p1f = p1.reshape(B * N, 3)
    p1f16 = _pad16(p1f)

    td = params["enc1_td"]
    y_pre, st = _dense(x0, td["lin"]["W"], td["lin"]["b"])
    s, t = _bn_scale_shift(td["bn"], st[0], st[1], B * N)

    idx1 = _knn_pallas(p1, p1, NSAMPLE[0])
    gidx1 = (idx1 + (jnp.arange(B, dtype=jnp.int32) * N)[:, None, None]).reshape(-1)
    gpos1 = _gather_rows(p1f16, gidx1)

    blk = params["enc1_blk"]
    h_pre, x1, st1 = _dense(y_pre, blk["l1"]["W"], blk["l1"]["b"],
                            pre=(s, t), emit_a=True)
    x1b = _pt_block_from(blk, p1f16, gpos1, x1, h_pre, st1, gidx1,
                         NSAMPLE[0], PLANES[0])

    sidx = _fps_pallas(p1)
    gsidx = (sidx + (jnp.arange(B, dtype=jnp.int32) * N)[:, None]).reshape(-1)
    p2f16 = _gather_rows(p1f16, gsidx)
    p2f = p2f16[:, :3]
    p2 = p2f.reshape(B, N2, 3)
    nidx = _knn_pallas(p2, p1, NSAMPLE[1])
    gnidx = (nidx + (jnp.arange(B, dtype=jnp.int32) * N)[:, None, None]).reshape(-1)
    gp = _gather_rows(p1f16, gnidx)
    gx = _gather_rows(x1b, gnidx)
    td = params["enc2_td"]
    f_pre, st = _td2_feat_kernel(gp, gx, p2f16, td["lin"]["W"][:3],
                                 td["lin"]["W"][3:], td["lin"]["b"], NSAMPLE[1])
    s, t = _bn_scale_shift(td["bn"], st[0], st[1], B * N2 * NSAMPLE[1])
    x2 = _td2_max_kernel(f_pre, s, t, NSAMPLE[1])

    idx2 = _knn_pallas(p2, p2, NSAMPLE[1])
    gidx2 = (idx2 + (jnp.arange(B, dtype=jnp.int32) * N2)[:, None, None]).reshape(-1)
    gpos2 = _gather_rows(p2f16, gidx2)

    x2b = _pt_block(params["enc2_blk"], p2f16, gpos2, x2, gidx2,
                    NSAMPLE[1], PLANES[1])

    up = params["dec2_up"]
    h2_pre, st = _dec2_pre_kernel(x2b, up["l2"]["W"], up["l2"]["b"],
                                  up["l1"]["W"][:PLANES[1]],
                                  up["l1"]["W"][PLANES[1]:], up["l1"]["b"])
    s, t = _bn_scale_shift(up["bn1"], st[0], st[1], B * N2)
    h2 = _ew_bnrelu(h2_pre, s, t)
    x2d = _pt_block(params["dec2_blk"], p2f16, gpos2, h2, gidx2,
                    NSAMPLE[1], PLANES[1])

    up = params["dec1_up"]
    a_pre, sta = _dense(x1b, up["l1"]["W"], up["l1"]["b"])
    sa, ta = _bn_scale_shift(up["bn1"], sta[0], sta[1], B * N)
    b_pre, stb = _dense(x2d, up["l2"]["W"], up["l2"]["b"])
    sb, tb = _bn_scale_shift(up["bn2"], stb[0], stb[1], B * N2)
    bfeat = _ew_bnrelu(b_pre, sb, tb)

    iidx = _knn_pallas(p1, p2, 3)
    iidx4 = jnp.concatenate([iidx, iidx[:, :, :1]], axis=-1)
    giidx = (iidx4 + (jnp.arange(B, dtype=jnp.int32) * N2)[:, None, None]
             ).reshape(-1)
    gpi = _gather_rows(p2f16, giidx)
    gxi = _gather_rows(bfeat, giidx)
    h1 = _interp_kernel(a_pre, sa, ta, gpi, p1f16, gxi)

    x1d = _pt_block(params["dec1_blk"], p1f16, gpos1, h1, gidx1,
                    NSAMPLE[0], PLANES[0])

    ch, eh = params["cls_head"], params["edge_head"]
    Wcat = jnp.concatenate([ch["l1"]["W"], eh["l1"]["W"]], axis=1)
    bcat = jnp.concatenate([ch["l1"]["b"], eh["l1"]["b"]])
    hh_pre, st = _dense(x1d, Wcat, bcat)
    sA, tA = _bn_scale_shift(ch["bn"], st[0][:, :32], st[1][:, :32], B * N)
    sB, tB = _bn_scale_shift(eh["bn"], st[0][:, 32:], st[1][:, 32:], B * N)
    s = jnp.concatenate([sA, sB])
    t = jnp.concatenate([tA, tB])
    cls, edge = _heads_kernel(hh_pre, s, t, ch["l2"]["W"], ch["l2"]["b"],
                              eh["l2"]["W"], eh["l2"]["b"])
    cls = jnp.transpose(cls.reshape(B, N, K_CLS), (0, 2, 1))
    edge = jnp.transpose(edge.reshape(B, N, 2), (0, 2, 1))
    return (cls, edge)


def _pt_block_from(p, posf16, gpos16, x, h_pre, st, idx, ns, c):
    M = x.shape[0]
    s, t = _bn_scale_shift(p["bn1"], st[0], st[1], M)
    tr = p["tr"]
    Wqkv = jnp.concatenate([tr["q"]["W"], tr["k"]["W"], tr["v"]["W"]], axis=1)
    bqkv = jnp.concatenate([tr["q"]["b"], tr["k"]["b"], tr["v"]["b"]])
    (qkv,) = _dense(h_pre, Wqkv, bqkv, pre=(s, t), want_stats=False)
    q, kf, v = qkv[:, :c], qkv[:, c:2 * c], qkv[:, 2 * c:]
    tt, st = _pt_layer(tr, posf16, gpos16, q, kf, v, idx, ns, c)
    s, t = _bn_scale_shift(p["bn2"], st[0], st[1], M)
    h3_pre, st = _dense(tt, p["l3"]["W"], p["l3"]["b"], pre=(s, t))
    s, t = _bn_scale_shift(p["bn3"], st[0], st[1], M)
    return _ew_bnrelu(h3_pre, s, t, add=x)


def kernel(inputs, params):
    return _forward(inputs, params)


# speedup vs baseline: 10.1085x; 1.1437x over previous
import functools
import jax
import jax.numpy as jnp
import numpy as np
from jax.experimental import pallas as pl
from jax.experimental.pallas import tpu as pltpu
from jax.experimental.pallas import tpu_sc as plsc

B, C, N = 2, 6, 4096
K_CLS = 13
PLANES = [32, 64]
NSAMPLE = [8, 16]
SHARE = 8
N2 = N // 4
EPS = 1e-5
NEG_BIG = 3.0e38


def _bn_scale_shift(p, s1, s2, count):
    s1 = s1.reshape(-1)
    s2 = s2.reshape(-1)
    m = s1 / count
    v = s2 / count - m * m
    s = p["g"] * jax.lax.rsqrt(v + EPS)
    t = p["b"] - m * s
    return s, t


def _row2(a):
    return a.reshape(1, -1)


def _gather_rows(table, idx):
    R, D = table.shape
    Rout = idx.shape[0]
    NW = 32
    assert Rout % NW == 0 and D % 16 == 0
    per_w = Rout // NW
    ch = min(128, per_w)
    assert per_w % ch == 0
    n_chunks = per_w // ch

    mesh = plsc.VectorSubcoreMesh(core_axis_name="c", subcore_axis_name="s")

    @functools.partial(
        pl.kernel, mesh=mesh,
        out_type=jax.ShapeDtypeStruct((Rout, D), jnp.float32),
        compiler_params=pltpu.CompilerParams(use_tc_tiling_on_sc=False),
        scratch_types=[
            pltpu.VMEM((per_w,), jnp.int32),
            pltpu.VMEM((ch, D), jnp.float32),
            pltpu.VMEM((ch, D), jnp.float32),
            pltpu.SemaphoreType.DMA,
            pltpu.SemaphoreType.DMA,
        ],
    )
    def k(table_hbm, idx_hbm, out_hbm, idx_v, rows0, rows1, sem0, sem1):
        wid = jax.lax.axis_index("s") * 2 + jax.lax.axis_index("c")
        base = wid * per_w
        pltpu.sync_copy(idx_hbm.at[pl.ds(base, per_w)], idx_v)
        bufs = [(rows0, sem0), (rows1, sem1)]
        cps = [None, None]
        for ci in range(n_chunks):
            rv, sem = bufs[ci % 2]
            cps[ci % 2] = pltpu.async_copy(
                table_hbm.at[idx_v.at[pl.ds(ci * ch, ch)]], rv, sem)
            if ci > 0:
                pv, psem = bufs[(ci - 1) % 2]
                cps[(ci - 1) % 2].wait()
                pltpu.sync_copy(pv, out_hbm.at[pl.ds(base + (ci - 1) * ch, ch)])
        lv, lsem = bufs[(n_chunks - 1) % 2]
        cps[(n_chunks - 1) % 2].wait()
        pltpu.sync_copy(lv, out_hbm.at[pl.ds(base + (n_chunks - 1) * ch, ch)])

    return k(table, idx)


def _dense(X, W, b, pre=None, want_stats=True, emit_a=False, bm=2048):
    M, din = X.shape
    dout = W.shape[1]
    bm = min(bm, M)
    grid = M // bm
    assert M % bm == 0

    def body(*refs):
        i = pl.program_id(0)
        ir = iter(refs)
        x_ref = next(ir)
        w_ref = next(ir)
        b_ref = next(ir)
        if pre is not None:
            s_ref = next(ir)
            t_ref = next(ir)
        y_ref = next(ir)
        a_ref = next(ir) if emit_a else None
        if want_stats:
            s1_ref = next(ir)
            s2_ref = next(ir)
        x = x_ref[...]
        if pre is not None:
            x = jnp.maximum(x * s_ref[...] + t_ref[...], 0.0)
        if emit_a:
            a_ref[...] = x
        y = jnp.dot(x, w_ref[...], preferred_element_type=jnp.float32) + b_ref[...]
        y_ref[...] = y
        if want_stats:
            @pl.when(i == 0)
            def _():
                s1_ref[...] = jnp.zeros_like(s1_ref)
                s2_ref[...] = jnp.zeros_like(s2_ref)
            s1_ref[...] += jnp.sum(y, 0, keepdims=True)
            s2_ref[...] += jnp.sum(y * y, 0, keepdims=True)

    in_specs = [
        pl.BlockSpec((bm, din), lambda i: (i, 0)),
        pl.BlockSpec((din, dout), lambda i: (0, 0)),
        pl.BlockSpec((1, dout), lambda i: (0, 0)),
    ]
    args = [X, W, _row2(b)]
    if pre is not None:
        in_specs += [pl.BlockSpec((1, din), lambda i: (0, 0))] * 2
        args += [_row2(pre[0]), _row2(pre[1])]
    out_specs = [pl.BlockSpec((bm, dout), lambda i: (i, 0))]
    out_shapes = [jax.ShapeDtypeStruct((M, dout), jnp.float32)]
    if emit_a:
        out_specs.append(pl.BlockSpec((bm, din), lambda i: (i, 0)))
        out_shapes.append(jax.ShapeDtypeStruct((M, din), jnp.float32))
    if want_stats:
        out_specs += [pl.BlockSpec((1, dout), lambda i: (0, 0))] * 2
        out_shapes += [jax.ShapeDtypeStruct((1, dout), jnp.float32)] * 2
    outs = pl.pallas_call(
        body, grid=(grid,), in_specs=in_specs, out_specs=out_specs,
        out_shape=out_shapes)(*args)
    outs = list(outs)
    y = outs.pop(0)
    a = outs.pop(0) if emit_a else None
    st = (outs[0], outs[1]) if want_stats else None
    res = [y]
    if emit_a:
        res.append(a)
    if want_stats:
        res.append(st)
    return res


def _ew_bnrelu(X, s, t, add=None, bm=2048):
    M, d = X.shape
    bm = min(bm, M)

    def body(*refs):
        if add is not None:
            x_ref, s_ref, t_ref, a_ref, o_ref = refs
        else:
            x_ref, s_ref, t_ref, o_ref = refs
        y = x_ref[...] * s_ref[...] + t_ref[...]
        if add is not None:
            y = y + a_ref[...]
        o_ref[...] = jnp.maximum(y, 0.0)

    in_specs = [pl.BlockSpec((bm, d), lambda i: (i, 0)),
                pl.BlockSpec((1, d), lambda i: (0, 0)),
                pl.BlockSpec((1, d), lambda i: (0, 0))]
    args = [X, _row2(s), _row2(t)]
    if add is not None:
        in_specs.append(pl.BlockSpec((bm, d), lambda i: (i, 0)))
        args.append(add)
    return pl.pallas_call(
        body, grid=(M // bm,), in_specs=in_specs,
        out_specs=pl.BlockSpec((bm, d), lambda i: (i, 0)),
        out_shape=jax.ShapeDtypeStruct((M, d), jnp.float32))(*args)


def _knn_pallas(qpos, rpos, k, bq=256):
    Bq, Mq, _ = qpos.shape
    Mr = rpos.shape[1]
    qp = jnp.concatenate([qpos, jnp.zeros((Bq, Mq, 13), jnp.float32)], -1)
    rt = jnp.transpose(rpos, (0, 2, 1))
    rt = jnp.concatenate([rt, jnp.zeros((Bq, 5, Mr), jnp.float32)], 1)

    def body(q_ref, r_ref, idx_ref):
        q = q_ref[0]
        r = r_ref[0]
        q3 = q[:, :3]
        r3 = r[:3, :]
        qq = jnp.sum(q3 * q3, 1, keepdims=True)
        rr = jnp.sum(r3 * r3, 0, keepdims=True)
        cross = jnp.dot(q3, r3, preferred_element_type=jnp.float32)
        d = qq - 2.0 * cross + rr
        iota_r = jax.lax.broadcasted_iota(jnp.int32, (bq, Mr), 1)
        for j in range(k):
            m = jnp.min(d, 1, keepdims=True)
            am = jnp.min(jnp.where(d == m, iota_r, Mr), 1, keepdims=True)
            idx_ref[0, :, pl.ds(j, 1)] = am
            d = jnp.where(iota_r == am, NEG_BIG, d)

    return pl.pallas_call(
        body, grid=(Bq, Mq // bq),
        in_specs=[pl.BlockSpec((1, bq, 16), lambda b, i: (b, i, 0)),
                  pl.BlockSpec((1, 8, Mr), lambda b, i: (b, 0, 0))],
        out_specs=pl.BlockSpec((1, bq, k), lambda b, i: (b, i, 0)),
        out_shape=jax.ShapeDtypeStruct((Bq, Mq, k), jnp.int32))(qp, rt)


def _fps_pallas(p1):
    SUB, LN = 8, N // 8
    RS = B * SUB
    pos = jnp.transpose(p1, (0, 2, 1)).reshape(B, 3, SUB, LN)
    pos = jnp.transpose(pos, (1, 0, 2, 3)).reshape(3, RS, LN)
    posrow = jnp.concatenate(
        [p1, jnp.zeros((B, N, 5), jnp.float32)], -1)
    LO = N2 // 8

    def _cat(vals, shape):
        return jnp.concatenate([jnp.broadcast_to(v, shape) for v in vals], 0)

    def body(p_ref, pr_ref, o_ref):
        ii = (jax.lax.broadcasted_iota(jnp.int32, (RS, LN), 0) % SUB * LN
              + jax.lax.broadcasted_iota(jnp.int32, (RS, LN), 1))
        ii_out = (jax.lax.broadcasted_iota(jnp.int32, (RS, LO), 0) % 8 * LO
                  + jax.lax.broadcasted_iota(jnp.int32, (RS, LO), 1))
        xr = p_ref[0]
        yr = p_ref[1]
        zr = p_ref[2]

        def step(i, st):
            dist, idx_arr, lasts = st
            rows = [pr_ref[b, pl.ds(lasts[b], 1), :] for b in range(B)]
            lx = _cat([r[:, 0:1] for r in rows], (SUB, 1))
            ly = _cat([r[:, 1:2] for r in rows], (SUB, 1))
            lz = _cat([r[:, 2:3] for r in rows], (SUB, 1))
            d = (xr - lx) ** 2 + (yr - ly) ** 2 + (zr - lz) ** 2
            dist = jnp.minimum(dist, d)
            rmax = jnp.max(dist, axis=1, keepdims=True)
            ms = [jnp.max(rmax[b * SUB:(b + 1) * SUB], axis=0, keepdims=True)
                  for b in range(B)]
            cand = jnp.where(dist == _cat(ms, (SUB, 1)), ii, N)
            rmin = jnp.min(cand, axis=1, keepdims=True)
            gs = [jnp.min(rmin[b * SUB:(b + 1) * SUB], axis=0, keepdims=True)
                  for b in range(B)]
            idx_arr = jnp.where(ii_out == i, _cat(gs, (SUB, LO)), idx_arr)
            return dist, idx_arr, tuple(g[0, 0] for g in gs)

        dist0 = jnp.full((RS, LN), 1e10, jnp.float32)
        idx0 = jnp.zeros((RS, LO), jnp.int32)
        _, idx_arr, _ = jax.lax.fori_loop(
            1, N2, step, (dist0, idx0, tuple(0 for _ in range(B))))
        for b in range(B):
            o_ref[b] = idx_arr[b * SUB:(b + 1) * SUB]

    out = pl.pallas_call(
        body,
        in_specs=[pl.BlockSpec(pos.shape, lambda: (0, 0, 0)),
                  pl.BlockSpec(posrow.shape, lambda: (0, 0, 0))],
        out_specs=pl.BlockSpec((B, 8, LO), lambda: (0, 0, 0)),
        out_shape=jax.ShapeDtypeStruct((B, 8, LO), jnp.int32))(pos, posrow)
    return out.reshape(B, N2)


def _pe_pre_kernel(gpos, posf, W, b, ns, bm=512):
    M = posf.shape[0]
    bm = min(bm, M)
    Wp = jnp.zeros((16, 16), jnp.float32).at[:3, :3].set(W)
    bp = jnp.zeros((16,), jnp.float32).at[:3].set(b)

    def body(g_ref, p_ref, w_ref, b_ref, o_ref, s1_ref, s2_ref):
        i = pl.program_id(0)
        g = g_ref[...].reshape(bm, ns, 16)
        p = p_ref[...]
        pr = g - p[:, None, :]
        pr = pr.reshape(bm * ns, 16)
        y = jnp.dot(pr, w_ref[...], preferred_element_type=jnp.float32) + b_ref[...]
        o_ref[...] = y
        @pl.when(i == 0)
        def _():
            s1_ref[...] = jnp.zeros_like(s1_ref)
            s2_ref[...] = jnp.zeros_like(s2_ref)
        s1_ref[...] += jnp.sum(y, 0, keepdims=True)
        s2_ref[...] += jnp.sum(y * y, 0, keepdims=True)

    outs = pl.pallas_call(
        body, grid=(M // bm,),
        in_specs=[pl.BlockSpec((bm * ns, 16), lambda i: (i, 0)),
                  pl.BlockSpec((bm, 16), lambda i: (i, 0)),
                  pl.BlockSpec((16, 16), lambda i: (0, 0)),
                  pl.BlockSpec((1, 16), lambda i: (0, 0))],
        out_specs=[pl.BlockSpec((bm * ns, 16), lambda i: (i, 0)),
                   pl.BlockSpec((1, 16), lambda i: (0, 0)),
                   pl.BlockSpec((1, 16), lambda i: (0, 0))],
        out_shape=[jax.ShapeDtypeStruct((M * ns, 16), jnp.float32),
                   jax.ShapeDtypeStruct((1, 16), jnp.float32),
                   jax.ShapeDtypeStruct((1, 16), jnp.float32)])(
        gpos, posf, Wp, _row2(bp))
    return outs[0], (outs[1][:, :3], outs[2][:, :3])


def _pe_w_kernel(pe_pre, x_k, q, s, t, W2, b2, ns, c, bm=512):
    M = q.shape[0]
    bm = min(bm, M)
    sp = jnp.zeros((16,), jnp.float32).at[:3].set(s)
    tp = jnp.zeros((16,), jnp.float32).at[:3].set(t)
    W2p = jnp.zeros((16, c), jnp.float32).at[:3, :].set(W2)

    def body(pp_ref, xk_ref, q_ref, s_ref, t_ref, w_ref, b_ref,
             pe_ref, wr_ref, s1_ref, s2_ref):
        i = pl.program_id(0)
        a = jnp.maximum(pp_ref[...] * s_ref[...] + t_ref[...], 0.0)
        pe = jnp.dot(a, w_ref[...], preferred_element_type=jnp.float32) + b_ref[...]
        pe_ref[...] = pe
        w = (xk_ref[...].reshape(bm, ns, c) - q_ref[...][:, None, :]).reshape(
            bm * ns, c) + pe
        wr_ref[...] = w
        @pl.when(i == 0)
        def _():
            s1_ref[...] = jnp.zeros_like(s1_ref)
            s2_ref[...] = jnp.zeros_like(s2_ref)
        s1_ref[...] += jnp.sum(w, 0, keepdims=True)
        s2_ref[...] += jnp.sum(w * w, 0, keepdims=True)

    outs = pl.pallas_call(
        body, grid=(M // bm,),
        in_specs=[pl.BlockSpec((bm * ns, 16), lambda i: (i, 0)),
                  pl.BlockSpec((bm * ns, c), lambda i: (i, 0)),
                  pl.BlockSpec((bm, c), lambda i: (i, 0)),
                  pl.BlockSpec((1, 16), lambda i: (0, 0)),
                  pl.BlockSpec((1, 16), lambda i: (0, 0)),
                  pl.BlockSpec((16, c), lambda i: (0, 0)),
                  pl.BlockSpec((1, c), lambda i: (0, 0))],
        out_specs=[pl.BlockSpec((bm * ns, c), lambda i: (i, 0)),
                   pl.BlockSpec((bm * ns, c), lambda i: (i, 0)),
                   pl.BlockSpec((1, c), lambda i: (0, 0)),
                   pl.BlockSpec((1, c), lambda i: (0, 0))],
        out_shape=[jax.ShapeDtypeStruct((M * ns, c), jnp.float32),
                   jax.ShapeDtypeStruct((M * ns, c), jnp.float32),
                   jax.ShapeDtypeStruct((1, c), jnp.float32),
                   jax.ShapeDtypeStruct((1, c), jnp.float32)])(
        pe_pre, x_k, q, _row2(sp), _row2(tp), W2p, _row2(b2))
    return outs[0], outs[1], (outs[2], outs[3])


def _attn_out_kernel(w1, x_v, pe, s, t, W2, b2, ns, c, bm=512):
    M = x_v.shape[0] // ns
    bm = min(bm, M)
    cs = c // SHARE

    def body(w1_ref, xv_ref, pe_ref, s_ref, t_ref, w_ref, b_ref,
             o_ref, s1_ref, s2_ref):
        i = pl.program_id(0)
        a = jnp.maximum(w1_ref[...] * s_ref[...] + t_ref[...], 0.0)
        w2 = jnp.dot(a, w_ref[...], preferred_element_type=jnp.float32) + b_ref[...]
        w3 = w2.reshape(bm, ns, cs)
        m = jnp.max(w3, axis=1, keepdims=True)
        e = jnp.exp(w3 - m)
        sm = e / jnp.sum(e, axis=1, keepdims=True)
        smf = jnp.concatenate([sm] * SHARE, axis=-1)
        xvpe = (xv_ref[...] + pe_ref[...]).reshape(bm, ns, c)
        out = jnp.sum(xvpe * smf, axis=1)
        o_ref[...] = out
        @pl.when(i == 0)
        def _():
            s1_ref[...] = jnp.zeros_like(s1_ref)
            s2_ref[...] = jnp.zeros_like(s2_ref)
        s1_ref[...] += jnp.sum(out, 0, keepdims=True)
        s2_ref[...] += jnp.sum(out * out, 0, keepdims=True)

    outs = pl.pallas_call(
        body, grid=(M // bm,),
        in_specs=[pl.BlockSpec((bm * ns, cs), lambda i: (i, 0)),
                  pl.BlockSpec((bm * ns, c), lambda i: (i, 0)),
                  pl.BlockSpec((bm * ns, c), lambda i: (i, 0)),
                  pl.BlockSpec((1, cs), lambda i: (0, 0)),
                  pl.BlockSpec((1, cs), lambda i: (0, 0)),
                  pl.BlockSpec((cs, cs), lambda i: (0, 0)),
                  pl.BlockSpec((1, cs), lambda i: (0, 0))],
        out_specs=[pl.BlockSpec((bm, c), lambda i: (i, 0)),
                   pl.BlockSpec((1, c), lambda i: (0, 0)),
                   pl.BlockSpec((1, c), lambda i: (0, 0))],
        out_shape=[jax.ShapeDtypeStruct((M, c), jnp.float32),
                   jax.ShapeDtypeStruct((1, c), jnp.float32),
                   jax.ShapeDtypeStruct((1, c), jnp.float32)])(
        w1, x_v, pe, _row2(s), _row2(t), W2, _row2(b2))
    return outs[0], (outs[1], outs[2])


def _td2_feat_kernel(gp, gx, p2f, Wrel, Wx, b, ns, bm=256):
    M = p2f.shape[0]
    bm = min(bm, M)
    dout = Wx.shape[1]
    Wr = jnp.zeros((16, dout), jnp.float32).at[:3, :].set(Wrel)

    def body(gp_ref, gx_ref, p_ref, wr_ref, wx_ref, b_ref, o_ref, s1_ref, s2_ref):
        i = pl.program_id(0)
        rel = (gp_ref[...].reshape(bm, ns, 16) - p_ref[...][:, None, :]).reshape(
            bm * ns, 16)
        y = (jnp.dot(rel, wr_ref[...], preferred_element_type=jnp.float32)
             + jnp.dot(gx_ref[...], wx_ref[...], preferred_element_type=jnp.float32)
             + b_ref[...])
        o_ref[...] = y
        @pl.when(i == 0)
        def _():
            s1_ref[...] = jnp.zeros_like(s1_ref)
            s2_ref[...] = jnp.zeros_like(s2_ref)
        s1_ref[...] += jnp.sum(y, 0, keepdims=True)
        s2_ref[...] += jnp.sum(y * y, 0, keepdims=True)

    din = gx.shape[1]
    outs = pl.pallas_call(
        body, grid=(M // bm,),
        in_specs=[pl.BlockSpec((bm * ns, 16), lambda i: (i, 0)),
                  pl.BlockSpec((bm * ns, din), lambda i: (i, 0)),
                  pl.BlockSpec((bm, 16), lambda i: (i, 0)),
                  pl.BlockSpec((16, dout), lambda i: (0, 0)),
                  pl.BlockSpec((din, dout), lambda i: (0, 0)),
                  pl.BlockSpec((1, dout), lambda i: (0, 0))],
        out_specs=[pl.BlockSpec((bm * ns, dout), lambda i: (i, 0)),
                   pl.BlockSpec((1, dout), lambda i: (0, 0)),
                   pl.BlockSpec((1, dout), lambda i: (0, 0))],
        out_shape=[jax.ShapeDtypeStruct((M * ns, dout), jnp.float32),
                   jax.ShapeDtypeStruct((1, dout), jnp.float32),
                   jax.ShapeDtypeStruct((1, dout), jnp.float32)])(
        gp, gx, p2f, Wr, Wx, _row2(b))
    return outs[0], (outs[1], outs[2])


def _td2_max_kernel(f_pre, s, t, ns, bm=256):
    Mns, d = f_pre.shape
    M = Mns // ns
    bm = min(bm, M)

    def body(f_ref, s_ref, t_ref, o_ref):
        a = jnp.maximum(f_ref[...] * s_ref[...] + t_ref[...], 0.0)
        o_ref[...] = jnp.max(a.reshape(bm, ns, d), axis=1)

    return pl.pallas_call(
        body, grid=(M // bm,),
        in_specs=[pl.BlockSpec((bm * ns, d), lambda i: (i, 0)),
                  pl.BlockSpec((1, d), lambda i: (0, 0)),
                  pl.BlockSpec((1, d), lambda i: (0, 0))],
        out_specs=pl.BlockSpec((bm, d), lambda i: (i, 0)),
        out_shape=jax.ShapeDtypeStruct((M, d), jnp.float32))(
        f_pre, _row2(s), _row2(t))


def _dec2_pre_kernel(x2b, W2, b2, W1a, W1b, b1):
    c = x2b.shape[1]

    def body(x_ref, w2_ref, b2_ref, wa_ref, wb_ref, b1_ref, o_ref, s1_ref, s2_ref):
        s1_ref[...] = jnp.zeros_like(s1_ref)
        s2_ref[...] = jnp.zeros_like(s2_ref)
        for bb in range(B):
            xb = x_ref[pl.ds(bb * N2, N2), :]
            gm = jnp.sum(xb, 0, keepdims=True) / N2
            g2 = jnp.maximum(jnp.dot(gm, w2_ref[...],
                                     preferred_element_type=jnp.float32)
                             + b2_ref[...], 0.0)
            y = (jnp.dot(xb, wa_ref[...], preferred_element_type=jnp.float32)
                 + jnp.dot(g2, wb_ref[...], preferred_element_type=jnp.float32)
                 + b1_ref[...])
            o_ref[pl.ds(bb * N2, N2), :] = y
            s1_ref[...] += jnp.sum(y, 0, keepdims=True)
            s2_ref[...] += jnp.sum(y * y, 0, keepdims=True)

    outs = pl.pallas_call(
        body,
        in_specs=[pl.BlockSpec(x2b.shape, lambda: (0, 0))] +
                 [pl.BlockSpec(a.shape, lambda: (0, 0)) for a in
                  (W2, _row2(b2), W1a, W1b, _row2(b1))],
        out_specs=[pl.BlockSpec((B * N2, c), lambda: (0, 0)),
                   pl.BlockSpec((1, c), lambda: (0, 0)),
                   pl.BlockSpec((1, c), lambda: (0, 0))],
        out_shape=[jax.ShapeDtypeStruct((B * N2, c), jnp.float32),
                   jax.ShapeDtypeStruct((1, c), jnp.float32),
                   jax.ShapeDtypeStruct((1, c), jnp.float32)])(
        x2b, W2, _row2(b2), W1a, W1b, _row2(b1))
    return outs[0], (outs[1], outs[2])


def _interp_kernel(a_pre, sa, ta, gpi, p1f, gxi, bm=1024):
    M, c = a_pre.shape
    bm = min(bm, M)

    def body(a_ref, s_ref, t_ref, gp_ref, p_ref, gx_ref, o_ref):
        a = jnp.maximum(a_ref[...] * s_ref[...] + t_ref[...], 0.0)
        gp = gp_ref[...].reshape(bm, 4, 16)[:, :3, :3]
        diff = gp - p_ref[...][:, None, :3]
        d = jnp.sqrt(jnp.sum(diff * diff, axis=-1)) + 1e-8
        w = 1.0 / d
        w = w / jnp.sum(w, -1, keepdims=True)
        gx = gx_ref[...].reshape(bm, 4, c)[:, :3, :]
        o_ref[...] = a + jnp.sum(gx * w[:, :, None], axis=1)

    return pl.pallas_call(
        body, grid=(M // bm,),
        in_specs=[pl.BlockSpec((bm, c), lambda i: (i, 0)),
                  pl.BlockSpec((1, c), lambda i: (0, 0)),
                  pl.BlockSpec((1, c), lambda i: (0, 0)),
                  pl.BlockSpec((bm * 4, 16), lambda i: (i, 0)),
                  pl.BlockSpec((bm, 16), lambda i: (i, 0)),
                  pl.BlockSpec((bm * 4, c), lambda i: (i, 0))],
        out_specs=pl.BlockSpec((bm, c), lambda i: (i, 0)),
        out_shape=jax.ShapeDtypeStruct((M, c), jnp.float32))(
        a_pre, _row2(sa), _row2(ta), gpi, p1f, gxi)


def _heads_kernel(hh_pre, s, t, Wc, bc, We, be, bm=2048):
    M = hh_pre.shape[0]
    bm = min(bm, M)

    def body(h_ref, s_ref, t_ref, wc_ref, bc_ref, we_ref, be_ref, c_ref, e_ref):
        h = jnp.maximum(h_ref[...] * s_ref[...] + t_ref[...], 0.0)
        c_ref[...] = jnp.dot(h[:, :32], wc_ref[...],
                             preferred_element_type=jnp.float32) + bc_ref[...]
        e_ref[...] = jnp.dot(h[:, 32:], we_ref[...],
                             preferred_element_type=jnp.float32) + be_ref[...]

    return pl.pallas_call(
        body, grid=(M // bm,),
        in_specs=[pl.BlockSpec((bm, 64), lambda i: (i, 0)),
                  pl.BlockSpec((1, 64), lambda i: (0, 0)),
                  pl.BlockSpec((1, 64), lambda i: (0, 0)),
                  pl.BlockSpec((32, K_CLS), lambda i: (0, 0)),
                  pl.BlockSpec((1, K_CLS), lambda i: (0, 0)),
                  pl.BlockSpec((32, 2), lambda i: (0, 0)),
                  pl.BlockSpec((1, 2), lambda i: (0, 0))],
        out_specs=[pl.BlockSpec((bm, K_CLS), lambda i: (i, 0)),
                   pl.BlockSpec((bm, 2), lambda i: (i, 0))],
        out_shape=[jax.ShapeDtypeStruct((M, K_CLS), jnp.float32),
                   jax.ShapeDtypeStruct((M, 2), jnp.float32)])(
        hh_pre, _row2(s), _row2(t), Wc, _row2(bc), We, _row2(be))


def _pt_layer(p, posf16, gpos16, q, kf, v, idx, ns, c):
    M = q.shape[0]
    pe_pre, st = _pe_pre_kernel(gpos16, posf16, p["p1"]["W"], p["p1"]["b"], ns)
    s, t = _bn_scale_shift(p["pbn"], st[0], st[1], M * ns)

    kv = jnp.concatenate([kf, v], axis=1)
    gkv = _gather_rows(kv, idx)
    x_k = gkv[:, :c]
    x_v = gkv[:, c:]

    pe, w_raw, st = _pe_w_kernel(pe_pre, x_k, q, s, t,
                                 p["p2"]["W"], p["p2"]["b"], ns, c)
    s, t = _bn_scale_shift(p["wbn1"], st[0], st[1], M * ns)
    w1, st = _dense(w_raw, p["w1"]["W"], p["w1"]["b"], pre=(s, t))
    s, t = _bn_scale_shift(p["wbn2"], st[0], st[1], M * ns)
    out, st = _attn_out_kernel(w1, x_v, pe, s, t,
                               p["w2"]["W"], p["w2"]["b"], ns, c)
    return out, st


def _pt_block(p, posf16, gpos16, x, idx, ns, c):
    h_pre, st = _dense(x, p["l1"]["W"], p["l1"]["b"])
    return _pt_block_from(p, posf16, gpos16, x, h_pre, st, idx, ns, c)


def _pad16(a):
    return jnp.concatenate([a, jnp.zeros(a.shape[:-1] + (16 - a.shape[-1],),
                                         jnp.float32)], -1)


def _forward(inputs, params):
    pxo = jnp.transpose(inputs, (0, 2, 1))
    x0 = pxo.reshape(B * N, C)
    p1 = pxo[:, :, :3]
    p1f = p1.reshape(B * N, 3)
    p1f16 = _pad16(p1f)

    td = params["enc1_td"]
    y_pre, st = _dense(x0, td["lin"]["W"], td["lin"]["b"])
    s, t = _bn_scale_shift(td["bn"], st[0], st[1], B * N)

    idx1 = _knn_pallas(p1, p1, NSAMPLE[0])
    gidx1 = (idx1 + (jnp.arange(B, dtype=jnp.int32) * N)[:, None, None]).reshape(-1)
    gpos1 = _gather_rows(p1f16, gidx1)

    blk = params["enc1_blk"]
    h_pre, x1, st1 = _dense(y_pre, blk["l1"]["W"], blk["l1"]["b"],
                            pre=(s, t), emit_a=True)
    x1b = _pt_block_from(blk, p1f16, gpos1, x1, h_pre, st1, gidx1,
                         NSAMPLE[0], PLANES[0])

    sidx = _fps_pallas(p1)
    gsidx = (sidx + (jnp.arange(B, dtype=jnp.int32) * N)[:, None]).reshape(-1)
    p2f16 = _gather_rows(p1f16, gsidx)
    p2f = p2f16[:, :3]
    p2 = p2f.reshape(B, N2, 3)
    nidx = _knn_pallas(p2, p1, NSAMPLE[1])
    gnidx = (nidx + (jnp.arange(B, dtype=jnp.int32) * N)[:, None, None]).reshape(-1)
    gp = _gather_rows(p1f16, gnidx)
    gx = _gather_rows(x1b, gnidx)
    td = params["enc2_td"]
    f_pre, st = _td2_feat_kernel(gp, gx, p2f16, td["lin"]["W"][:3],
                                 td["lin"]["W"][3:], td["lin"]["b"], NSAMPLE[1])
    s, t = _bn_scale_shift(td["bn"], st[0], st[1], B * N2 * NSAMPLE[1])
    x2 = _td2_max_kernel(f_pre, s, t, NSAMPLE[1])

    idx2 = _knn_pallas(p2, p2, NSAMPLE[1])
    gidx2 = (idx2 + (jnp.arange(B, dtype=jnp.int32) * N2)[:, None, None]).reshape(-1)
    gpos2 = _gather_rows(p2f16, gidx2)

    x2b = _pt_block(params["enc2_blk"], p2f16, gpos2, x2, gidx2,
                    NSAMPLE[1], PLANES[1])

    up = params["dec2_up"]
    h2_pre, st = _dec2_pre_kernel(x2b, up["l2"]["W"], up["l2"]["b"],
                                  up["l1"]["W"][:PLANES[1]],
                                  up["l1"]["W"][PLANES[1]:], up["l1"]["b"])
    s, t = _bn_scale_shift(up["bn1"], st[0], st[1], B * N2)
    h2 = _ew_bnrelu(h2_pre, s, t)
    x2d = _pt_block(params["dec2_blk"], p2f16, gpos2, h2, gidx2,
                    NSAMPLE[1], PLANES[1])

    up = params["dec1_up"]
    a_pre, sta = _dense(x1b, up["l1"]["W"], up["l1"]["b"])
    sa, ta = _bn_scale_shift(up["bn1"], sta[0], sta[1], B * N)
    b_pre, stb = _dense(x2d, up["l2"]["W"], up["l2"]["b"])
    sb, tb = _bn_scale_shift(up["bn2"], stb[0], stb[1], B * N2)
    bfeat = _ew_bnrelu(b_pre, sb, tb)

    iidx = _knn_pallas(p1, p2, 3)
    iidx4 = jnp.concatenate([iidx, iidx[:, :, :1]], axis=-1)
    giidx = (iidx4 + (jnp.arange(B, dtype=jnp.int32) * N2)[:, None, None]
             ).reshape(-1)
    gpi = _gather_rows(p2f16, giidx)
    gxi = _gather_rows(bfeat, giidx)
    h1 = _interp_kernel(a_pre, sa, ta, gpi, p1f16, gxi)

    x1d = _pt_block(params["dec1_blk"], p1f16, gpos1, h1, gidx1,
                    NSAMPLE[0], PLANES[0])

    ch, eh = params["cls_head"], params["edge_head"]
    Wcat = jnp.concatenate([ch["l1"]["W"], eh["l1"]["W"]], axis=1)
    bcat = jnp.concatenate([ch["l1"]["b"], eh["l1"]["b"]])
    hh_pre, st = _dense(x1d, Wcat, bcat)
    sA, tA = _bn_scale_shift(ch["bn"], st[0][:, :32], st[1][:, :32], B * N)
    sB, tB = _bn_scale_shift(eh["bn"], st[0][:, 32:], st[1][:, 32:], B * N)
    s = jnp.concatenate([sA, sB])
    t = jnp.concatenate([tA, tB])
    cls, edge = _heads_kernel(hh_pre, s, t, ch["l2"]["W"], ch["l2"]["b"],
                              eh["l2"]["W"], eh["l2"]["b"])
    cls = jnp.transpose(cls.reshape(B, N, K_CLS), (0, 2, 1))
    edge = jnp.transpose(edge.reshape(B, N, 2), (0, 2, 1))
    return (cls, edge)


def _pt_block_from(p, posf16, gpos16, x, h_pre, st, idx, ns, c):
    M = x.shape[0]
    s, t = _bn_scale_shift(p["bn1"], st[0], st[1], M)
    tr = p["tr"]
    Wqkv = jnp.concatenate([tr["q"]["W"], tr["k"]["W"], tr["v"]["W"]], axis=1)
    bqkv = jnp.concatenate([tr["q"]["b"], tr["k"]["b"], tr["v"]["b"]])
    (qkv,) = _dense(h_pre, Wqkv, bqkv, pre=(s, t), want_stats=False)
    q, kf, v = qkv[:, :c], qkv[:, c:2 * c], qkv[:, 2 * c:]
    tt, st = _pt_layer(tr, posf16, gpos16, q, kf, v, idx, ns, c)
    s, t = _bn_scale_shift(p["bn2"], st[0], st[1], M)
    h3_pre, st = _dense(tt, p["l3"]["W"], p["l3"]["b"], pre=(s, t))
    s, t = _bn_scale_shift(p["bn3"], st[0], st[1], M)
    return _ew_bnrelu(h3_pre, s, t, add=x)


def kernel(inputs, params):
    return _forward(inputs, params)


# speedup vs baseline: 10.5967x; 1.0483x over previous
import functools
import jax
import jax.numpy as jnp
import numpy as np
from jax.experimental import pallas as pl
from jax.experimental.pallas import tpu as pltpu
from jax.experimental.pallas import tpu_sc as plsc

B, C, N = 2, 6, 4096
K_CLS = 13
PLANES = [32, 64]
NSAMPLE = [8, 16]
SHARE = 8
N2 = N // 4
EPS = 1e-5
NEG_BIG = 3.0e38


def _bn_scale_shift(p, s1, s2, count):
    s1 = s1.reshape(-1)
    s2 = s2.reshape(-1)
    m = s1 / count
    v = s2 / count - m * m
    s = p["g"] * jax.lax.rsqrt(v + EPS)
    t = p["b"] - m * s
    return s, t


def _row2(a):
    return a.reshape(1, -1)


def _gather_rows(table, idx, wide=False):
    R, D0 = table.shape
    if wide:
        D = 128
        if D0 < D:
            table = jnp.concatenate(
                [table, jnp.zeros((R, D - D0), jnp.float32)], -1)
        cp = {}
    else:
        D = D0
        cp = {"compiler_params": pltpu.CompilerParams(use_tc_tiling_on_sc=False)}
    Rout = idx.shape[0]
    NW = 32
    assert Rout % NW == 0 and D % 16 == 0
    per_w = Rout // NW
    ch = min(128, per_w)
    assert per_w % ch == 0
    n_chunks = per_w // ch

    mesh = plsc.VectorSubcoreMesh(core_axis_name="c", subcore_axis_name="s")

    @functools.partial(
        pl.kernel, mesh=mesh,
        out_type=jax.ShapeDtypeStruct((Rout, D), jnp.float32),
        **cp,
        scratch_types=[
            pltpu.VMEM((per_w,), jnp.int32),
            pltpu.VMEM((ch, D), jnp.float32),
            pltpu.VMEM((ch, D), jnp.float32),
            pltpu.SemaphoreType.DMA,
            pltpu.SemaphoreType.DMA,
        ],
    )
    def k(table_hbm, idx_hbm, out_hbm, idx_v, rows0, rows1, sem0, sem1):
        wid = jax.lax.axis_index("s") * 2 + jax.lax.axis_index("c")
        base = wid * per_w
        pltpu.sync_copy(idx_hbm.at[pl.ds(base, per_w)], idx_v)
        bufs = [(rows0, sem0), (rows1, sem1)]
        cps = [None, None]
        for ci in range(n_chunks):
            rv, sem = bufs[ci % 2]
            cps[ci % 2] = pltpu.async_copy(
                table_hbm.at[idx_v.at[pl.ds(ci * ch, ch)]], rv, sem)
            if ci > 0:
                pv, psem = bufs[(ci - 1) % 2]
                cps[(ci - 1) % 2].wait()
                pltpu.sync_copy(pv, out_hbm.at[pl.ds(base + (ci - 1) * ch, ch)])
        lv, lsem = bufs[(n_chunks - 1) % 2]
        cps[(n_chunks - 1) % 2].wait()
        pltpu.sync_copy(lv, out_hbm.at[pl.ds(base + (n_chunks - 1) * ch, ch)])

    return k(table, idx)


def _dense(X, W, b, pre=None, want_stats=True, emit_a=False, bm=2048):
    M, din = X.shape
    dout = W.shape[1]
    bm = min(bm, M)
    grid = M // bm
    assert M % bm == 0

    def body(*refs):
        i = pl.program_id(0)
        ir = iter(refs)
        x_ref = next(ir)
        w_ref = next(ir)
        b_ref = next(ir)
        if pre is not None:
            s_ref = next(ir)
            t_ref = next(ir)
        y_ref = next(ir)
        a_ref = next(ir) if emit_a else None
        if want_stats:
            s1_ref = next(ir)
            s2_ref = next(ir)
        x = x_ref[...]
        if pre is not None:
            x = jnp.maximum(x * s_ref[...] + t_ref[...], 0.0)
        if emit_a:
            a_ref[...] = x
        y = jnp.dot(x, w_ref[...], preferred_element_type=jnp.float32) + b_ref[...]
        y_ref[...] = y
        if want_stats:
            @pl.when(i == 0)
            def _():
                s1_ref[...] = jnp.zeros_like(s1_ref)
                s2_ref[...] = jnp.zeros_like(s2_ref)
            s1_ref[...] += jnp.sum(y, 0, keepdims=True)
            s2_ref[...] += jnp.sum(y * y, 0, keepdims=True)

    in_specs = [
        pl.BlockSpec((bm, din), lambda i: (i, 0)),
        pl.BlockSpec((din, dout), lambda i: (0, 0)),
        pl.BlockSpec((1, dout), lambda i: (0, 0)),
    ]
    args = [X, W, _row2(b)]
    if pre is not None:
        in_specs += [pl.BlockSpec((1, din), lambda i: (0, 0))] * 2
        args += [_row2(pre[0]), _row2(pre[1])]
    out_specs = [pl.BlockSpec((bm, dout), lambda i: (i, 0))]
    out_shapes = [jax.ShapeDtypeStruct((M, dout), jnp.float32)]
    if emit_a:
        out_specs.append(pl.BlockSpec((bm, din), lambda i: (i, 0)))
        out_shapes.append(jax.ShapeDtypeStruct((M, din), jnp.float32))
    if want_stats:
        out_specs += [pl.BlockSpec((1, dout), lambda i: (0, 0))] * 2
        out_shapes += [jax.ShapeDtypeStruct((1, dout), jnp.float32)] * 2
    outs = pl.pallas_call(
        body, grid=(grid,), in_specs=in_specs, out_specs=out_specs,
        out_shape=out_shapes)(*args)
    outs = list(outs)
    y = outs.pop(0)
    a = outs.pop(0) if emit_a else None
    st = (outs[0], outs[1]) if want_stats else None
    res = [y]
    if emit_a:
        res.append(a)
    if want_stats:
        res.append(st)
    return res


def _ew_bnrelu(X, s, t, add=None, bm=2048):
    M, d = X.shape
    bm = min(bm, M)

    def body(*refs):
        if add is not None:
            x_ref, s_ref, t_ref, a_ref, o_ref = refs
        else:
            x_ref, s_ref, t_ref, o_ref = refs
        y = x_ref[...] * s_ref[...] + t_ref[...]
        if add is not None:
            y = y + a_ref[...]
        o_ref[...] = jnp.maximum(y, 0.0)

    in_specs = [pl.BlockSpec((bm, d), lambda i: (i, 0)),
                pl.BlockSpec((1, d), lambda i: (0, 0)),
                pl.BlockSpec((1, d), lambda i: (0, 0))]
    args = [X, _row2(s), _row2(t)]
    if add is not None:
        in_specs.append(pl.BlockSpec((bm, d), lambda i: (i, 0)))
        args.append(add)
    return pl.pallas_call(
        body, grid=(M // bm,), in_specs=in_specs,
        out_specs=pl.BlockSpec((bm, d), lambda i: (i, 0)),
        out_shape=jax.ShapeDtypeStruct((M, d), jnp.float32))(*args)


def _knn_pallas(qpos, rpos, k, bq=256):
    Bq, Mq, _ = qpos.shape
    Mr = rpos.shape[1]
    qp = jnp.concatenate([qpos, jnp.zeros((Bq, Mq, 13), jnp.float32)], -1)
    rt = jnp.transpose(rpos, (0, 2, 1))
    rt = jnp.concatenate([rt, jnp.zeros((Bq, 5, Mr), jnp.float32)], 1)

    def body(q_ref, r_ref, idx_ref):
        q = q_ref[0]
        r = r_ref[0]
        q3 = q[:, :3]
        r3 = r[:3, :]
        qq = jnp.sum(q3 * q3, 1, keepdims=True)
        rr = jnp.sum(r3 * r3, 0, keepdims=True)
        cross = (q[:, 0:1] * r[0:1, :] + q[:, 1:2] * r[1:2, :]
                 + q[:, 2:3] * r[2:3, :])
        d = qq - 2.0 * cross + rr
        iota_r = jax.lax.broadcasted_iota(jnp.int32, (bq, Mr), 1)
        for j in range(k):
            m = jnp.min(d, 1, keepdims=True)
            am = jnp.min(jnp.where(d == m, iota_r, Mr), 1, keepdims=True)
            idx_ref[0, :, pl.ds(j, 1)] = am
            d = jnp.where(iota_r == am, NEG_BIG, d)

    return pl.pallas_call(
        body, grid=(Bq, Mq // bq),
        in_specs=[pl.BlockSpec((1, bq, 16), lambda b, i: (b, i, 0)),
                  pl.BlockSpec((1, 8, Mr), lambda b, i: (b, 0, 0))],
        out_specs=pl.BlockSpec((1, bq, k), lambda b, i: (b, i, 0)),
        out_shape=jax.ShapeDtypeStruct((Bq, Mq, k), jnp.int32))(qp, rt)


def _fps_pallas(p1):
    SUB, LN = 8, N // 8
    RS = B * SUB
    pos = jnp.transpose(p1, (0, 2, 1)).reshape(B, 3, SUB, LN)
    pos = jnp.transpose(pos, (1, 0, 2, 3)).reshape(3, RS, LN)
    posrow = jnp.concatenate(
        [p1, jnp.zeros((B, N, 5), jnp.float32)], -1)
    LO = N2 // 8

    def _cat(vals, shape):
        return jnp.concatenate([jnp.broadcast_to(v, shape) for v in vals], 0)

    def body(p_ref, pr_ref, o_ref):
        ii = (jax.lax.broadcasted_iota(jnp.int32, (RS, LN), 0) % SUB * LN
              + jax.lax.broadcasted_iota(jnp.int32, (RS, LN), 1))
        ii_out = (jax.lax.broadcasted_iota(jnp.int32, (RS, LO), 0) % 8 * LO
                  + jax.lax.broadcasted_iota(jnp.int32, (RS, LO), 1))
        xr = p_ref[0]
        yr = p_ref[1]
        zr = p_ref[2]

        def step(i, st):
            dist, idx_arr, lasts = st
            rows = [pr_ref[b, pl.ds(lasts[b], 1), :] for b in range(B)]
            lx = _cat([r[:, 0:1] for r in rows], (SUB, 1))
            ly = _cat([r[:, 1:2] for r in rows], (SUB, 1))
            lz = _cat([r[:, 2:3] for r in rows], (SUB, 1))
            d = (xr - lx) ** 2 + (yr - ly) ** 2 + (zr - lz) ** 2
            dist = jnp.minimum(dist, d)
            rmax = jnp.max(dist, axis=1, keepdims=True)
            ms = [jnp.max(rmax[b * SUB:(b + 1) * SUB], axis=0, keepdims=True)
                  for b in range(B)]
            cand = jnp.where(dist == _cat(ms, (SUB, 1)), ii, N)
            rmin = jnp.min(cand, axis=1, keepdims=True)
            gs = [jnp.min(rmin[b * SUB:(b + 1) * SUB], axis=0, keepdims=True)
                  for b in range(B)]
            idx_arr = jnp.where(ii_out == i, _cat(gs, (SUB, LO)), idx_arr)
            return dist, idx_arr, tuple(g[0, 0] for g in gs)

        dist0 = jnp.full((RS, LN), 1e10, jnp.float32)
        idx0 = jnp.zeros((RS, LO), jnp.int32)
        _, idx_arr, _ = jax.lax.fori_loop(
            1, N2, step, (dist0, idx0, tuple(0 for _ in range(B))))
        for b in range(B):
            o_ref[b] = idx_arr[b * SUB:(b + 1) * SUB]

    out = pl.pallas_call(
        body,
        in_specs=[pl.BlockSpec(pos.shape, lambda: (0, 0, 0)),
                  pl.BlockSpec(posrow.shape, lambda: (0, 0, 0))],
        out_specs=pl.BlockSpec((B, 8, LO), lambda: (0, 0, 0)),
        out_shape=jax.ShapeDtypeStruct((B, 8, LO), jnp.int32))(pos, posrow)
    return out.reshape(B, N2)


def _pe_pre_kernel(gpos, posf, W, b, ns, bm=512):
    M = posf.shape[0]
    bm = min(bm, M)
    Wp = jnp.zeros((16, 16), jnp.float32).at[:3, :3].set(W)
    bp = jnp.zeros((16,), jnp.float32).at[:3].set(b)

    def body(g_ref, p_ref, w_ref, b_ref, o_ref, s1_ref, s2_ref):
        i = pl.program_id(0)
        g = g_ref[...].reshape(bm, ns, 16)
        p = p_ref[...]
        pr = g - p[:, None, :]
        pr = pr.reshape(bm * ns, 16)
        y = jnp.dot(pr, w_ref[...], preferred_element_type=jnp.float32) + b_ref[...]
        o_ref[...] = y
        @pl.when(i == 0)
        def _():
            s1_ref[...] = jnp.zeros_like(s1_ref)
            s2_ref[...] = jnp.zeros_like(s2_ref)
        s1_ref[...] += jnp.sum(y, 0, keepdims=True)
        s2_ref[...] += jnp.sum(y * y, 0, keepdims=True)

    outs = pl.pallas_call(
        body, grid=(M // bm,),
        in_specs=[pl.BlockSpec((bm * ns, 16), lambda i: (i, 0)),
                  pl.BlockSpec((bm, 16), lambda i: (i, 0)),
                  pl.BlockSpec((16, 16), lambda i: (0, 0)),
                  pl.BlockSpec((1, 16), lambda i: (0, 0))],
        out_specs=[pl.BlockSpec((bm * ns, 16), lambda i: (i, 0)),
                   pl.BlockSpec((1, 16), lambda i: (0, 0)),
                   pl.BlockSpec((1, 16), lambda i: (0, 0))],
        out_shape=[jax.ShapeDtypeStruct((M * ns, 16), jnp.float32),
                   jax.ShapeDtypeStruct((1, 16), jnp.float32),
                   jax.ShapeDtypeStruct((1, 16), jnp.float32)])(
        gpos, posf, Wp, _row2(bp))
    return outs[0], (outs[1][:, :3], outs[2][:, :3])


def _pe_w_kernel(pe_pre, kv, q, s, t, W2, b2, ns, c, bm=512):
    M = q.shape[0]
    bm = min(bm, M)
    sp = jnp.zeros((16,), jnp.float32).at[:3].set(s)
    tp = jnp.zeros((16,), jnp.float32).at[:3].set(t)
    W2p = jnp.zeros((16, c), jnp.float32).at[:3, :].set(W2)

    def body(pp_ref, kv_ref, q_ref, s_ref, t_ref, w_ref, b_ref,
             pe_ref, wr_ref, s1_ref, s2_ref):
        i = pl.program_id(0)
        a = jnp.maximum(pp_ref[...] * s_ref[...] + t_ref[...], 0.0)
        pe = jnp.dot(a, w_ref[...], preferred_element_type=jnp.float32) + b_ref[...]
        pe_ref[...] = pe
        x_k = kv_ref[...][:, :c]
        w = (x_k.reshape(bm, ns, c) - q_ref[...][:, None, :]).reshape(
            bm * ns, c) + pe
        wr_ref[...] = w
        @pl.when(i == 0)
        def _():
            s1_ref[...] = jnp.zeros_like(s1_ref)
            s2_ref[...] = jnp.zeros_like(s2_ref)
        s1_ref[...] += jnp.sum(w, 0, keepdims=True)
        s2_ref[...] += jnp.sum(w * w, 0, keepdims=True)

    outs = pl.pallas_call(
        body, grid=(M // bm,),
        in_specs=[pl.BlockSpec((bm * ns, 16), lambda i: (i, 0)),
                  pl.BlockSpec((bm * ns, 128), lambda i: (i, 0)),
                  pl.BlockSpec((bm, c), lambda i: (i, 0)),
                  pl.BlockSpec((1, 16), lambda i: (0, 0)),
                  pl.BlockSpec((1, 16), lambda i: (0, 0)),
                  pl.BlockSpec((16, c), lambda i: (0, 0)),
                  pl.BlockSpec((1, c), lambda i: (0, 0))],
        out_specs=[pl.BlockSpec((bm * ns, c), lambda i: (i, 0)),
                   pl.BlockSpec((bm * ns, c), lambda i: (i, 0)),
                   pl.BlockSpec((1, c), lambda i: (0, 0)),
                   pl.BlockSpec((1, c), lambda i: (0, 0))],
        out_shape=[jax.ShapeDtypeStruct((M * ns, c), jnp.float32),
                   jax.ShapeDtypeStruct((M * ns, c), jnp.float32),
                   jax.ShapeDtypeStruct((1, c), jnp.float32),
                   jax.ShapeDtypeStruct((1, c), jnp.float32)])(
        pe_pre, kv, q, _row2(sp), _row2(tp), W2p, _row2(b2))
    return outs[0], outs[1], (outs[2], outs[3])


def _attn_out_kernel(w1, kv, pe, s, t, W2, b2, ns, c, bm=512):
    M = kv.shape[0] // ns
    bm = min(bm, M)
    cs = c // SHARE

    def body(w1_ref, kv_ref, pe_ref, s_ref, t_ref, w_ref, b_ref,
             o_ref, s1_ref, s2_ref):
        i = pl.program_id(0)
        a = jnp.maximum(w1_ref[...] * s_ref[...] + t_ref[...], 0.0)
        w2 = jnp.dot(a, w_ref[...], preferred_element_type=jnp.float32) + b_ref[...]
        w3 = w2.reshape(bm, ns, cs)
        m = jnp.max(w3, axis=1, keepdims=True)
        e = jnp.exp(w3 - m)
        sm = e / jnp.sum(e, axis=1, keepdims=True)
        smf = jnp.concatenate([sm] * SHARE, axis=-1)
        xvpe = (kv_ref[...][:, c:2 * c] + pe_ref[...]).reshape(bm, ns, c)
        out = jnp.sum(xvpe * smf, axis=1)
        o_ref[...] = out
        @pl.when(i == 0)
        def _():
            s1_ref[...] = jnp.zeros_like(s1_ref)
            s2_ref[...] = jnp.zeros_like(s2_ref)
        s1_ref[...] += jnp.sum(out, 0, keepdims=True)
        s2_ref[...] += jnp.sum(out * out, 0, keepdims=True)

    outs = pl.pallas_call(
        body, grid=(M // bm,),
        in_specs=[pl.BlockSpec((bm * ns, cs), lambda i: (i, 0)),
                  pl.BlockSpec((bm * ns, 128), lambda i: (i, 0)),
                  pl.BlockSpec((bm * ns, c), lambda i: (i, 0)),
                  pl.BlockSpec((1, cs), lambda i: (0, 0)),
                  pl.BlockSpec((1, cs), lambda i: (0, 0)),
                  pl.BlockSpec((cs, cs), lambda i: (0, 0)),
                  pl.BlockSpec((1, cs), lambda i: (0, 0))],
        out_specs=[pl.BlockSpec((bm, c), lambda i: (i, 0)),
                   pl.BlockSpec((1, c), lambda i: (0, 0)),
                   pl.BlockSpec((1, c), lambda i: (0, 0))],
        out_shape=[jax.ShapeDtypeStruct((M, c), jnp.float32),
                   jax.ShapeDtypeStruct((1, c), jnp.float32),
                   jax.ShapeDtypeStruct((1, c), jnp.float32)])(
        w1, kv, pe, _row2(s), _row2(t), W2, _row2(b2))
    return outs[0], (outs[1], outs[2])


def _td2_feat_kernel(gp, gx, p2f, Wrel, Wx, b, ns, bm=256):
    M = p2f.shape[0]
    bm = min(bm, M)
    dout = Wx.shape[1]
    din = Wx.shape[0]
    Wr = jnp.zeros((16, dout), jnp.float32).at[:3, :].set(Wrel)

    def body(gp_ref, gx_ref, p_ref, wr_ref, wx_ref, b_ref, o_ref, s1_ref, s2_ref):
        i = pl.program_id(0)
        rel = (gp_ref[...].reshape(bm, ns, 16) - p_ref[...][:, None, :]).reshape(
            bm * ns, 16)
        y = (jnp.dot(rel, wr_ref[...], preferred_element_type=jnp.float32)
             + jnp.dot(gx_ref[...][:, :din], wx_ref[...],
                       preferred_element_type=jnp.float32)
             + b_ref[...])
        o_ref[...] = y
        @pl.when(i == 0)
        def _():
            s1_ref[...] = jnp.zeros_like(s1_ref)
            s2_ref[...] = jnp.zeros_like(s2_ref)
        s1_ref[...] += jnp.sum(y, 0, keepdims=True)
        s2_ref[...] += jnp.sum(y * y, 0, keepdims=True)

    outs = pl.pallas_call(
        body, grid=(M // bm,),
        in_specs=[pl.BlockSpec((bm * ns, 16), lambda i: (i, 0)),
                  pl.BlockSpec((bm * ns, 128), lambda i: (i, 0)),
                  pl.BlockSpec((bm, 16), lambda i: (i, 0)),
                  pl.BlockSpec((16, dout), lambda i: (0, 0)),
                  pl.BlockSpec((din, dout), lambda i: (0, 0)),
                  pl.BlockSpec((1, dout), lambda i: (0, 0))],
        out_specs=[pl.BlockSpec((bm * ns, dout), lambda i: (i, 0)),
                   pl.BlockSpec((1, dout), lambda i: (0, 0)),
                   pl.BlockSpec((1, dout), lambda i: (0, 0))],
        out_shape=[jax.ShapeDtypeStruct((M * ns, dout), jnp.float32),
                   jax.ShapeDtypeStruct((1, dout), jnp.float32),
                   jax.ShapeDtypeStruct((1, dout), jnp.float32)])(
        gp, gx, p2f, Wr, Wx, _row2(b))
    return outs[0], (outs[1], outs[2])


def _td2_max_kernel(f_pre, s, t, ns, bm=256):
    Mns, d = f_pre.shape
    M = Mns // ns
    bm = min(bm, M)

    def body(f_ref, s_ref, t_ref, o_ref):
        a = jnp.maximum(f_ref[...] * s_ref[...] + t_ref[...], 0.0)
        o_ref[...] = jnp.max(a.reshape(bm, ns, d), axis=1)

    return pl.pallas_call(
        body, grid=(M // bm,),
        in_specs=[pl.BlockSpec((bm * ns, d), lambda i: (i, 0)),
                  pl.BlockSpec((1, d), lambda i: (0, 0)),
                  pl.BlockSpec((1, d), lambda i: (0, 0))],
        out_specs=pl.BlockSpec((bm, d), lambda i: (i, 0)),
        out_shape=jax.ShapeDtypeStruct((M, d), jnp.float32))(
        f_pre, _row2(s), _row2(t))


def _dec2_pre_kernel(x2b, W2, b2, W1a, W1b, b1):
    c = x2b.shape[1]

    def body(x_ref, w2_ref, b2_ref, wa_ref, wb_ref, b1_ref, o_ref, s1_ref, s2_ref):
        s1_ref[...] = jnp.zeros_like(s1_ref)
        s2_ref[...] = jnp.zeros_like(s2_ref)
        for bb in range(B):
            xb = x_ref[pl.ds(bb * N2, N2), :]
            gm = jnp.sum(xb, 0, keepdims=True) / N2
            g2 = jnp.maximum(jnp.dot(gm, w2_ref[...],
                                     preferred_element_type=jnp.float32)
                             + b2_ref[...], 0.0)
            y = (jnp.dot(xb, wa_ref[...], preferred_element_type=jnp.float32)
                 + jnp.dot(g2, wb_ref[...], preferred_element_type=jnp.float32)
                 + b1_ref[...])
            o_ref[pl.ds(bb * N2, N2), :] = y
            s1_ref[...] += jnp.sum(y, 0, keepdims=True)
            s2_ref[...] += jnp.sum(y * y, 0, keepdims=True)

    outs = pl.pallas_call(
        body,
        in_specs=[pl.BlockSpec(x2b.shape, lambda: (0, 0))] +
                 [pl.BlockSpec(a.shape, lambda: (0, 0)) for a in
                  (W2, _row2(b2), W1a, W1b, _row2(b1))],
        out_specs=[pl.BlockSpec((B * N2, c), lambda: (0, 0)),
                   pl.BlockSpec((1, c), lambda: (0, 0)),
                   pl.BlockSpec((1, c), lambda: (0, 0))],
        out_shape=[jax.ShapeDtypeStruct((B * N2, c), jnp.float32),
                   jax.ShapeDtypeStruct((1, c), jnp.float32),
                   jax.ShapeDtypeStruct((1, c), jnp.float32)])(
        x2b, W2, _row2(b2), W1a, W1b, _row2(b1))
    return outs[0], (outs[1], outs[2])


def _interp_kernel(a_pre, sa, ta, gpi, p1f, gxi, bm=1024):
    M, c = a_pre.shape
    bm = min(bm, M)

    def body(a_ref, s_ref, t_ref, gp_ref, p_ref, gx_ref, o_ref):
        a = jnp.maximum(a_ref[...] * s_ref[...] + t_ref[...], 0.0)
        gp = gp_ref[...].reshape(bm, 4, 16)[:, :3, :3]
        diff = gp - p_ref[...][:, None, :3]
        d = jnp.sqrt(jnp.sum(diff * diff, axis=-1)) + 1e-8
        w = 1.0 / d
        w = w / jnp.sum(w, -1, keepdims=True)
        gx = gx_ref[...][:, :c].reshape(bm, 4, c)[:, :3, :]
        o_ref[...] = a + jnp.sum(gx * w[:, :, None], axis=1)

    return pl.pallas_call(
        body, grid=(M // bm,),
        in_specs=[pl.BlockSpec((bm, c), lambda i: (i, 0)),
                  pl.BlockSpec((1, c), lambda i: (0, 0)),
                  pl.BlockSpec((1, c), lambda i: (0, 0)),
                  pl.BlockSpec((bm * 4, 16), lambda i: (i, 0)),
                  pl.BlockSpec((bm, 16), lambda i: (i, 0)),
                  pl.BlockSpec((bm * 4, 128), lambda i: (i, 0))],
        out_specs=pl.BlockSpec((bm, c), lambda i: (i, 0)),
        out_shape=jax.ShapeDtypeStruct((M, c), jnp.float32))(
        a_pre, _row2(sa), _row2(ta), gpi, p1f, gxi)


def _heads_kernel(hh_pre, s, t, Wc, bc, We, be, bm=2048):
    M = hh_pre.shape[0]
    bm = min(bm, M)

    def body(h_ref, s_ref, t_ref, wc_ref, bc_ref, we_ref, be_ref, c_ref, e_ref):
        h = jnp.maximum(h_ref[...] * s_ref[...] + t_ref[...], 0.0)
        c_ref[...] = jnp.dot(h[:, :32], wc_ref[...],
                             preferred_element_type=jnp.float32) + bc_ref[...]
        e_ref[...] = jnp.dot(h[:, 32:], we_ref[...],
                             preferred_element_type=jnp.float32) + be_ref[...]

    return pl.pallas_call(
        body, grid=(M // bm,),
        in_specs=[pl.BlockSpec((bm, 64), lambda i: (i, 0)),
                  pl.BlockSpec((1, 64), lambda i: (0, 0)),
                  pl.BlockSpec((1, 64), lambda i: (0, 0)),
                  pl.BlockSpec((32, K_CLS), lambda i: (0, 0)),
                  pl.BlockSpec((1, K_CLS), lambda i: (0, 0)),
                  pl.BlockSpec((32, 2), lambda i: (0, 0)),
                  pl.BlockSpec((1, 2), lambda i: (0, 0))],
        out_specs=[pl.BlockSpec((bm, K_CLS), lambda i: (i, 0)),
                   pl.BlockSpec((bm, 2), lambda i: (i, 0))],
        out_shape=[jax.ShapeDtypeStruct((M, K_CLS), jnp.float32),
                   jax.ShapeDtypeStruct((M, 2), jnp.float32)])(
        hh_pre, _row2(s), _row2(t), Wc, _row2(bc), We, _row2(be))


def _pt_layer(p, posf16, gpos16, q, kf, v, idx, ns, c):
    M = q.shape[0]
    pe_pre, st = _pe_pre_kernel(gpos16, posf16, p["p1"]["W"], p["p1"]["b"], ns)
    s, t = _bn_scale_shift(p["pbn"], st[0], st[1], M * ns)

    kv = jnp.concatenate([kf, v], axis=1)
    gkv = _gather_rows(kv, idx, wide=True)

    pe, w_raw, st = _pe_w_kernel(pe_pre, gkv, q, s, t,
                                 p["p2"]["W"], p["p2"]["b"], ns, c)
    s, t = _bn_scale_shift(p["wbn1"], st[0], st[1], M * ns)
    w1, st = _dense(w_raw, p["w1"]["W"], p["w1"]["b"], pre=(s, t))
    s, t = _bn_scale_shift(p["wbn2"], st[0], st[1], M * ns)
    out, st = _attn_out_kernel(w1, gkv, pe, s, t,
                               p["w2"]["W"], p["w2"]["b"], ns, c)
    return out, st


def _pt_block(p, posf16, gpos16, x, idx, ns, c):
    h_pre, st = _dense(x, p["l1"]["W"], p["l1"]["b"])
    return _pt_block_from(p, posf16, gpos16, x, h_pre, st, idx, ns, c)


def _pad16(a):
    return jnp.concatenate([a, jnp.zeros(a.shape[:-1] + (16 - a.shape[-1],),
                                         jnp.float32)], -1)


def _forward(inputs, params):
    pxo = jnp.transpose(inputs, (0, 2, 1))
    x0 = pxo.reshape(B * N, C)
    p1 = pxo[:, :, :3]
    p1f = p1.reshape(B * N, 3)
    p1f16 = _pad16(p1f)

    td = params["enc1_td"]
    y_pre, st = _dense(x0, td["lin"]["W"], td["lin"]["b"])
    s, t = _bn_scale_shift(td["bn"], st[0], st[1], B * N)

    idx1 = _knn_pallas(p1, p1, NSAMPLE[0])
    gidx1 = (idx1 + (jnp.arange(B, dtype=jnp.int32) * N)[:, None, None]).reshape(-1)
    gpos1 = _gather_rows(p1f16, gidx1)

    blk = params["enc1_blk"]
    h_pre, x1, st1 = _dense(y_pre, blk["l1"]["W"], blk["l1"]["b"],
                            pre=(s, t), emit_a=True)
    x1b = _pt_block_from(blk, p1f16, gpos1, x1, h_pre, st1, gidx1,
                         NSAMPLE[0], PLANES[0])

    sidx = _fps_pallas(p1)
    gsidx = (sidx + (jnp.arange(B, dtype=jnp.int32) * N)[:, None]).reshape(-1)
    p2f16 = _gather_rows(p1f16, gsidx)
    p2f = p2f16[:, :3]
    p2 = p2f.reshape(B, N2, 3)
    nidx = _knn_pallas(p2, p1, NSAMPLE[1])
    gnidx = (nidx + (jnp.arange(B, dtype=jnp.int32) * N)[:, None, None]).reshape(-1)
    gp = _gather_rows(p1f16, gnidx)
    gx = _gather_rows(x1b, gnidx, wide=True)
    td = params["enc2_td"]
    f_pre, st = _td2_feat_kernel(gp, gx, p2f16, td["lin"]["W"][:3],
                                 td["lin"]["W"][3:], td["lin"]["b"], NSAMPLE[1])
    s, t = _bn_scale_shift(td["bn"], st[0], st[1], B * N2 * NSAMPLE[1])
    x2 = _td2_max_kernel(f_pre, s, t, NSAMPLE[1])

    idx2 = _knn_pallas(p2, p2, NSAMPLE[1])
    gidx2 = (idx2 + (jnp.arange(B, dtype=jnp.int32) * N2)[:, None, None]).reshape(-1)
    gpos2 = _gather_rows(p2f16, gidx2)

    x2b = _pt_block(params["enc2_blk"], p2f16, gpos2, x2, gidx2,
                    NSAMPLE[1], PLANES[1])

    up = params["dec2_up"]
    h2_pre, st = _dec2_pre_kernel(x2b, up["l2"]["W"], up["l2"]["b"],
                                  up["l1"]["W"][:PLANES[1]],
                                  up["l1"]["W"][PLANES[1]:], up["l1"]["b"])
    s, t = _bn_scale_shift(up["bn1"], st[0], st[1], B * N2)
    h2 = _ew_bnrelu(h2_pre, s, t)
    x2d = _pt_block(params["dec2_blk"], p2f16, gpos2, h2, gidx2,
                    NSAMPLE[1], PLANES[1])

    up = params["dec1_up"]
    a_pre, sta = _dense(x1b, up["l1"]["W"], up["l1"]["b"])
    sa, ta = _bn_scale_shift(up["bn1"], sta[0], sta[1], B * N)
    b_pre, stb = _dense(x2d, up["l2"]["W"], up["l2"]["b"])
    sb, tb = _bn_scale_shift(up["bn2"], stb[0], stb[1], B * N2)
    bfeat = _ew_bnrelu(b_pre, sb, tb)

    iidx = _knn_pallas(p1, p2, 3)
    iidx4 = jnp.concatenate([iidx, iidx[:, :, :1]], axis=-1)
    giidx = (iidx4 + (jnp.arange(B, dtype=jnp.int32) * N2)[:, None, None]
             ).reshape(-1)
    gpi = _gather_rows(p2f16, giidx)
    gxi = _gather_rows(bfeat, giidx, wide=True)
    h1 = _interp_kernel(a_pre, sa, ta, gpi, p1f16, gxi)

    x1d = _pt_block(params["dec1_blk"], p1f16, gpos1, h1, gidx1,
                    NSAMPLE[0], PLANES[0])

    ch, eh = params["cls_head"], params["edge_head"]
    Wcat = jnp.concatenate([ch["l1"]["W"], eh["l1"]["W"]], axis=1)
    bcat = jnp.concatenate([ch["l1"]["b"], eh["l1"]["b"]])
    hh_pre, st = _dense(x1d, Wcat, bcat)
    sA, tA = _bn_scale_shift(ch["bn"], st[0][:, :32], st[1][:, :32], B * N)
    sB, tB = _bn_scale_shift(eh["bn"], st[0][:, 32:], st[1][:, 32:], B * N)
    s = jnp.concatenate([sA, sB])
    t = jnp.concatenate([tA, tB])
    cls, edge = _heads_kernel(hh_pre, s, t, ch["l2"]["W"], ch["l2"]["b"],
                              eh["l2"]["W"], eh["l2"]["b"])
    cls = jnp.transpose(cls.reshape(B, N, K_CLS), (0, 2, 1))
    edge = jnp.transpose(edge.reshape(B, N, 2), (0, 2, 1))
    return (cls, edge)


def _pt_block_from(p, posf16, gpos16, x, h_pre, st, idx, ns, c):
    M = x.shape[0]
    s, t = _bn_scale_shift(p["bn1"], st[0], st[1], M)
    tr = p["tr"]
    Wqkv = jnp.concatenate([tr["q"]["W"], tr["k"]["W"], tr["v"]["W"]], axis=1)
    bqkv = jnp.concatenate([tr["q"]["b"], tr["k"]["b"], tr["v"]["b"]])
    (qkv,) = _dense(h_pre, Wqkv, bqkv, pre=(s, t), want_stats=False)
    q, kf, v = qkv[:, :c], qkv[:, c:2 * c], qkv[:, 2 * c:]
    tt, st = _pt_layer(tr, posf16, gpos16, q, kf, v, idx, ns, c)
    s, t = _bn_scale_shift(p["bn2"], st[0], st[1], M)
    h3_pre, st = _dense(tt, p["l3"]["W"], p["l3"]["b"], pre=(s, t))
    s, t = _bn_scale_shift(p["bn3"], st[0], st[1], M)
    return _ew_bnrelu(h3_pre, s, t, add=x)


def kernel(inputs, params):
    return _forward(inputs, params)
